# Initial kernel scaffold; baseline (speedup 1.0000x reference)
import jax
import jax.numpy as jnp
from jax.experimental import pallas as pl


def kernel(x, edge_index, fi_w1, fi_b1, fi_g1, fi_be1, fi_w2, fi_b2, fi_g2, fi_be2, fo_w1, fo_b1, fo_g1, fo_be1, fo_w2, fo_b2, fo_g2, fo_be2, fp_w1, fp_b1, fp_g1, fp_be1):
    raise NotImplementedError("write your pallas kernel here")


# speedup vs baseline: 1.4559x; 1.4559x over previous
import functools

import jax
import jax.numpy as jnp
from jax import lax
from jax.experimental import pallas as pl
from jax.experimental.pallas import tpu as pltpu
from jax.experimental.pallas import tpu_sc as plsc

N = 10000
E = 160000
D = 256
EPS = 1e-5

NC = 2
NS = 16
NW = NC * NS

HC = 5120
ACC_R = 5248
NPAD = 2 * HC

CG = 40
GCH = (E // NW) // CG
CS = 80
SCH = (E // NS) // CS

TBE = 2000
TBN = 2000

def _sc_mesh():
    return plsc.VectorSubcoreMesh(core_axis_name="c", subcore_axis_name="s")


def _gather_body(x_hbm, src_hbm, dst_hbm, xu_hbm, xv_hbm, idx_v, rows_v, sem):
    wid = lax.axis_index("s") * NC + lax.axis_index("c")
    epw = E // NW
    base = pl.multiple_of(wid * epw, epw)
    for idx_hbm, out in ((src_hbm, xu_hbm), (dst_hbm, xv_hbm)):
        pltpu.sync_copy(idx_hbm.at[pl.ds(base, epw)], idx_v)

        def chunk(j, _):
            off = pl.multiple_of(j * CG, CG)
            pltpu.async_copy(
                x_hbm.at[idx_v.at[pl.ds(off, CG)]], rows_v, sem).wait()
            pltpu.sync_copy(rows_v, out.at[pl.ds(base + off, CG)])
            return 0

        lax.fori_loop(0, GCH, chunk, 0)


def _gather(x, src, dst):
    f = functools.partial(
        pl.kernel,
        mesh=_sc_mesh(),
        out_type=[
            jax.ShapeDtypeStruct((E, D), jnp.float32),
            jax.ShapeDtypeStruct((E, D), jnp.float32),
        ],
        scratch_types=[
            pltpu.VMEM((E // NW,), jnp.int32),
            pltpu.VMEM((CG, D), jnp.float32),
            pltpu.SemaphoreType.DMA,
        ],
    )(_gather_body)
    return f(x, src, dst)


def _zero_fill(buf, nrows, width):
    def zrow(r, _):
        for k in range(width // 16):
            buf[r, pl.ds(k * 16, 16)] = jnp.zeros((16,), jnp.float32)
        return 0
    lax.fori_loop(0, nrows, zrow, 0)


def _localize(idx_v, base_node):
    for k in range(CS // 16):
        v = idx_v[pl.ds(k * 16, 16)]
        loc = v - base_node
        ok = (loc >= 0) & (loc < HC)
        idx_v[pl.ds(k * 16, 16)] = jnp.where(ok, loc, HC)


SOUT = NPAD
H = 128


def _scatter_body(mi_hbm, mo_hbm, src_hbm, dst_hbm,
                  s_hbm, alo_sh, ahi_sh, idx_v, rlo_v, rhi_v, zb_v, sem):
    c = lax.axis_index("c")
    s = lax.axis_index("s")
    base_node = c * HC

    _zero_fill(zb_v, 40, H)
    stripe = pl.multiple_of(s * 328, 8)
    for acc in (alo_sh, ahi_sh):
        for q in range(8):
            pltpu.sync_copy(zb_v, acc.at[pl.ds(stripe + q * 40, 40)])
        pltpu.sync_copy(zb_v.at[pl.ds(0, 8)], acc.at[pl.ds(stripe + 320, 8)])

    plsc.subcore_barrier()

    ept = E // NS
    rbase = pl.multiple_of(s * ept, 8)

    def do_array(msg_hbm, idx_hbm):
        def chunk(j, _):
            off = pl.multiple_of(j * CS, CS)
            pltpu.sync_copy(idx_hbm.at[pl.ds(rbase + off, CS)], idx_v)
            _localize(idx_v, base_node)
            pltpu.sync_copy(msg_hbm.at[pl.ds(rbase + off, CS), pl.ds(0, H)],
                            rlo_v)
            pltpu.sync_copy(msg_hbm.at[pl.ds(rbase + off, CS), pl.ds(H, H)],
                            rhi_v)
            pltpu.async_copy(rlo_v, alo_sh.at[idx_v], sem, add=True).wait()
            pltpu.async_copy(rhi_v, ahi_sh.at[idx_v], sem, add=True).wait()
            return 0

        lax.fori_loop(0, SCH, chunk, 0)

    do_array(mi_hbm, dst_hbm)
    do_array(mo_hbm, src_hbm)

    plsc.subcore_barrier()

    dump = pl.multiple_of(s * 320, 8)
    obase = pl.multiple_of(c * HC + s * 320, 8)
    pltpu.sync_copy(alo_sh.at[pl.ds(dump, 320)],
                    s_hbm.at[pl.ds(obase, 320), pl.ds(0, H)])
    pltpu.sync_copy(ahi_sh.at[pl.ds(dump, 320)],
                    s_hbm.at[pl.ds(obase, 320), pl.ds(H, H)])


def _scatter(mi, mo, src, dst):
    f = functools.partial(
        pl.kernel,
        mesh=_sc_mesh(),
        out_type=jax.ShapeDtypeStruct((SOUT, D), jnp.float32),
        scratch_types=[
            pltpu.VMEM_SHARED((ACC_R, H), jnp.float32),
            pltpu.VMEM_SHARED((ACC_R, H), jnp.float32),
            pltpu.VMEM((CS,), jnp.int32),
            pltpu.VMEM((CS, H), jnp.float32),
            pltpu.VMEM((CS, H), jnp.float32),
            pltpu.VMEM((40, H), jnp.float32),
            pltpu.SemaphoreType.DMA,
        ],
    )(_scatter_body)
    return f(mi, mo, src, dst)


DACC_R = 5128


def _degree_body(src_hbm, dst_hbm, deg_hbm, dacc_sh, di_v, ones_v, zb_v, sem):
    c = lax.axis_index("c")
    s = lax.axis_index("s")
    base_node = c * HC

    _zero_fill(zb_v, 40, 128)
    stripe = pl.multiple_of(s * 320, 8)
    for q in range(8):
        pltpu.sync_copy(zb_v, dacc_sh.at[pl.ds(stripe + q * 40, 40)])

    @pl.when(s == 0)
    def _():
        pltpu.sync_copy(zb_v.at[pl.ds(0, 8)], dacc_sh.at[pl.ds(HC, 8)])

    def orow(r, _):
        for k in range(128 // 16):
            ones_v[r, pl.ds(k * 16, 16)] = jnp.ones((16,), jnp.float32)
        return 0
    lax.fori_loop(0, CS, orow, 0)

    plsc.subcore_barrier()

    ept = E // NS
    rbase = pl.multiple_of(s * ept, 8)

    def do_array(idx_hbm):
        def chunk(j, _):
            off = pl.multiple_of(j * CS, CS)
            pltpu.sync_copy(idx_hbm.at[pl.ds(rbase + off, CS)], di_v)
            _localize(di_v, base_node)
            pltpu.async_copy(ones_v, dacc_sh.at[di_v], sem, add=True).wait()
            return 0

        lax.fori_loop(0, SCH, chunk, 0)

    do_array(dst_hbm)
    do_array(src_hbm)

    plsc.subcore_barrier()

    obase = pl.multiple_of(c * HC + stripe, 8)
    pltpu.sync_copy(dacc_sh.at[pl.ds(stripe, 320)],
                    deg_hbm.at[pl.ds(obase, 320)])


def _degree(src, dst):
    f = functools.partial(
        pl.kernel,
        mesh=_sc_mesh(),
        out_type=jax.ShapeDtypeStruct((NPAD, 128), jnp.float32),
        scratch_types=[
            pltpu.VMEM_SHARED((DACC_R, 128), jnp.float32),
            pltpu.VMEM((CS,), jnp.int32),
            pltpu.VMEM((CS, 128), jnp.float32),
            pltpu.VMEM((40, 128), jnp.float32),
            pltpu.SemaphoreType.DMA,
        ],
    )(_degree_body)
    return f(src, dst)


def _k1_body(xu_ref, xv_ref, fiw_ref, fow_ref,
             h1i_ref, h1o_ref, si_ref, qi_ref, so_ref, qo_ref):
    xu = xu_ref[...]
    xv = xv_ref[...]
    h1i = (jnp.dot(xu, fiw_ref[:D, :], preferred_element_type=jnp.float32)
           + jnp.dot(xv, fiw_ref[D:, :], preferred_element_type=jnp.float32))
    h1o = (jnp.dot(xv, fow_ref[:D, :], preferred_element_type=jnp.float32)
           + jnp.dot(xu, fow_ref[D:, :], preferred_element_type=jnp.float32))
    h1i_ref[...] = h1i
    h1o_ref[...] = h1o

    @pl.when(pl.program_id(0) == 0)
    def _():
        si_ref[...] = jnp.zeros_like(si_ref)
        qi_ref[...] = jnp.zeros_like(qi_ref)
        so_ref[...] = jnp.zeros_like(so_ref)
        qo_ref[...] = jnp.zeros_like(qo_ref)

    si_ref[...] += jnp.sum(h1i, axis=0, keepdims=True)
    qi_ref[...] += jnp.sum(h1i * h1i, axis=0, keepdims=True)
    so_ref[...] += jnp.sum(h1o, axis=0, keepdims=True)
    qo_ref[...] += jnp.sum(h1o * h1o, axis=0, keepdims=True)


def _affine(sum_ref, sq_ref, g_ref, be_ref, n):
    m = sum_ref[...] * (1.0 / n)
    v = sq_ref[...] * (1.0 / n) - m * m
    a = g_ref[...] * lax.rsqrt(v + EPS)
    return a, be_ref[...] - m * a


def _k2_body(h1i_ref, h1o_ref, fiw2_ref, fow2_ref,
             si_ref, qi_ref, so_ref, qo_ref,
             gi_ref, bi_ref, go_ref, bo_ref,
             h2i_ref, h2o_ref, s2i_ref, q2i_ref, s2o_ref, q2o_ref):
    ai, ci = _affine(si_ref, qi_ref, gi_ref, bi_ref, float(E))
    ao, co = _affine(so_ref, qo_ref, go_ref, bo_ref, float(E))
    x2i = jnp.maximum(h1i_ref[...] * ai + ci, 0.0)
    x2o = jnp.maximum(h1o_ref[...] * ao + co, 0.0)
    h2i = jnp.dot(x2i, fiw2_ref[...], preferred_element_type=jnp.float32)
    h2o = jnp.dot(x2o, fow2_ref[...], preferred_element_type=jnp.float32)
    h2i_ref[...] = h2i
    h2o_ref[...] = h2o

    @pl.when(pl.program_id(0) == 0)
    def _():
        s2i_ref[...] = jnp.zeros_like(s2i_ref)
        q2i_ref[...] = jnp.zeros_like(q2i_ref)
        s2o_ref[...] = jnp.zeros_like(s2o_ref)
        q2o_ref[...] = jnp.zeros_like(q2o_ref)

    s2i_ref[...] += jnp.sum(h2i, axis=0, keepdims=True)
    q2i_ref[...] += jnp.sum(h2i * h2i, axis=0, keepdims=True)
    s2o_ref[...] += jnp.sum(h2o, axis=0, keepdims=True)
    q2o_ref[...] += jnp.sum(h2o * h2o, axis=0, keepdims=True)


def _k3_body(h2i_ref, h2o_ref,
             s2i_ref, q2i_ref, s2o_ref, q2o_ref,
             gi_ref, bi_ref, go_ref, bo_ref,
             mi_ref, mo_ref):
    ai, ci = _affine(s2i_ref, q2i_ref, gi_ref, bi_ref, float(E))
    ao, co = _affine(s2o_ref, q2o_ref, go_ref, bo_ref, float(E))
    mi_ref[...] = jnp.maximum(h2i_ref[...] * ai + ci, 0.0)
    mo_ref[...] = jnp.maximum(h2o_ref[...] * ao + co, 0.0)


def _k4_body(x_ref, s_ref, deg_ref, fpw_ref, g_ref, sp_ref, qp_ref):
    dv = jnp.maximum(deg_ref[...][:, 0:1], 1.0)
    h = x_ref[...] + s_ref[...] * (1.0 / dv)
    g = jnp.dot(h, fpw_ref[...], preferred_element_type=jnp.float32)
    g_ref[...] = g

    @pl.when(pl.program_id(0) == 0)
    def _():
        sp_ref[...] = jnp.zeros_like(sp_ref)
        qp_ref[...] = jnp.zeros_like(qp_ref)

    sp_ref[...] += jnp.sum(g, axis=0, keepdims=True)
    qp_ref[...] += jnp.sum(g * g, axis=0, keepdims=True)


def _k5_body(g_ref, sp_ref, qp_ref, gg_ref, bb_ref, x_ref):
    a, c = _affine(sp_ref, qp_ref, gg_ref, bb_ref, float(N))
    x_ref[...] = jnp.maximum(g_ref[...] * a + c, 0.0)


def _row_spec(tb, d):
    return pl.BlockSpec((tb, d), lambda i: (i, 0))


def _full_spec(shape):
    return pl.BlockSpec(shape, lambda i: tuple(0 for _ in shape))


_STAT = _full_spec((1, D))


def _k1(xu, xv, fiw1, fow1):
    ge = E // TBE
    return pl.pallas_call(
        _k1_body,
        grid=(ge,),
        in_specs=[_row_spec(TBE, D), _row_spec(TBE, D),
                  _full_spec((2 * D, D)), _full_spec((2 * D, D))],
        out_specs=[_row_spec(TBE, D), _row_spec(TBE, D),
                   _STAT, _STAT, _STAT, _STAT],
        out_shape=[jax.ShapeDtypeStruct((E, D), jnp.float32),
                   jax.ShapeDtypeStruct((E, D), jnp.float32)]
                  + [jax.ShapeDtypeStruct((1, D), jnp.float32)] * 4,
    )(xu, xv, fiw1, fow1)


def _k2(h1i, h1o, fiw2, fow2, si, qi, so, qo, gi, bi, go, bo):
    ge = E // TBE
    return pl.pallas_call(
        _k2_body,
        grid=(ge,),
        in_specs=[_row_spec(TBE, D), _row_spec(TBE, D),
                  _full_spec((D, D)), _full_spec((D, D))]
                 + [_STAT] * 8,
        out_specs=[_row_spec(TBE, D), _row_spec(TBE, D),
                   _STAT, _STAT, _STAT, _STAT],
        out_shape=[jax.ShapeDtypeStruct((E, D), jnp.float32),
                   jax.ShapeDtypeStruct((E, D), jnp.float32)]
                  + [jax.ShapeDtypeStruct((1, D), jnp.float32)] * 4,
    )(h1i, h1o, fiw2, fow2, si, qi, so, qo, gi, bi, go, bo)


def _k3(h2i, h2o, s2i, q2i, s2o, q2o, gi, bi, go, bo):
    ge = E // TBE
    return pl.pallas_call(
        _k3_body,
        grid=(ge,),
        in_specs=[_row_spec(TBE, D), _row_spec(TBE, D)] + [_STAT] * 8,
        out_specs=[_row_spec(TBE, D), _row_spec(TBE, D)],
        out_shape=[jax.ShapeDtypeStruct((E, D), jnp.float32),
                   jax.ShapeDtypeStruct((E, D), jnp.float32)],
    )(h2i, h2o, s2i, q2i, s2o, q2o, gi, bi, go, bo)


def _k4(x, s, deg, fpw):
    gn = N // TBN
    return pl.pallas_call(
        _k4_body,
        grid=(gn,),
        in_specs=[_row_spec(TBN, D), _row_spec(TBN, D),
                  _row_spec(TBN, 128), _full_spec((D, D))],
        out_specs=[_row_spec(TBN, D), _STAT, _STAT],
        out_shape=[jax.ShapeDtypeStruct((N, D), jnp.float32),
                   jax.ShapeDtypeStruct((1, D), jnp.float32),
                   jax.ShapeDtypeStruct((1, D), jnp.float32)],
    )(x, s, deg, fpw)


def _k5(g, sp, qp, gg, bb):
    gn = N // TBN
    return pl.pallas_call(
        _k5_body,
        grid=(gn,),
        in_specs=[_row_spec(TBN, D)] + [_STAT] * 4,
        out_specs=_row_spec(TBN, D),
        out_shape=jax.ShapeDtypeStruct((N, D), jnp.float32),
    )(g, sp, qp, gg, bb)


def kernel(x, edge_index,
           fi_w1, fi_b1, fi_g1, fi_be1, fi_w2, fi_b2, fi_g2, fi_be2,
           fo_w1, fo_b1, fo_g1, fo_be1, fo_w2, fo_b2, fo_g2, fo_be2,
           fp_w1, fp_b1, fp_g1, fp_be1):
    del fi_b1, fi_b2, fo_b1, fo_b2, fp_b1
    r = lambda t: t.reshape(1, D)
    gi1, bi1, gi2, bi2 = r(fi_g1), r(fi_be1), r(fi_g2), r(fi_be2)
    go1, bo1, go2, bo2 = r(fo_g1), r(fo_be1), r(fo_g2), r(fo_be2)
    gp, bp = r(fp_g1), r(fp_be1)

    src, dst = edge_index[0], edge_index[1]
    deg = _degree(src, dst)[:N]
    for step in range(2):
        xu, xv = _gather(x, src, dst)
        h1i, h1o, si, qi, so, qo = _k1(xu, xv, fi_w1, fo_w1)
        h2i, h2o, s2i, q2i, s2o, q2o = _k2(
            h1i, h1o, fi_w2, fo_w2, si, qi, so, qo, gi1, bi1, go1, bo1)
        mi, mo = _k3(h2i, h2o, s2i, q2i, s2o, q2o, gi2, bi2, go2, bo2)
        s_agg = _scatter(mi, mo, src, dst)
        g, sp, qp = _k4(x, s_agg[:N], deg, fp_w1)
        x = _k5(g, sp, qp, gp, bp)
    return x


# speedup vs baseline: 1.9826x; 1.3617x over previous
import functools

import jax
import jax.numpy as jnp
from jax import lax
from jax.experimental import pallas as pl
from jax.experimental.pallas import tpu as pltpu
from jax.experimental.pallas import tpu_sc as plsc

N = 10000
E = 160000
D = 256
EPS = 1e-5

NC = 2
NS = 16
NW = NC * NS

HC = 5120
ACC_R = 5248
NPAD = 2 * HC

CG = 40
GCH = (E // NW) // CG
CS = 80
SCH = (E // NS) // CS

TBE = 2000
TBN = 2000

def _sc_mesh():
    return plsc.VectorSubcoreMesh(core_axis_name="c", subcore_axis_name="s")


def _gather_body(x_hbm, src_hbm, dst_hbm, xu_hbm, xv_hbm,
                 idx_v, rows0_v, rows1_v, gs0, gs1, ws0, ws1):
    wid = lax.axis_index("s") * NC + lax.axis_index("c")
    epw = E // NW
    base = pl.multiple_of(wid * epw, epw)
    rows = (rows0_v, rows1_v)
    gsem = (gs0, gs1)
    wsem = (ws0, ws1)

    for idx_hbm, out in ((src_hbm, xu_hbm), (dst_hbm, xv_hbm)):
        pltpu.sync_copy(idx_hbm.at[pl.ds(base, epw)], idx_v)

        def gstart(b, j):
            off = pl.multiple_of(j * CG, 8)
            pltpu.async_copy(x_hbm.at[idx_v.at[pl.ds(off, CG)]],
                             rows[b], gsem[b])

        def gwait(b):
            pltpu.make_async_copy(x_hbm.at[idx_v.at[pl.ds(0, CG)]],
                                  rows[b], gsem[b]).wait()

        def wstart(b, j):
            off = pl.multiple_of(j * CG, 8)
            pltpu.async_copy(rows[b], out.at[pl.ds(base + off, CG)], wsem[b])

        def wwait(b):
            pltpu.make_async_copy(rows[b], out.at[pl.ds(base, CG)],
                                  wsem[b]).wait()

        gstart(0, 0)

        def pair(p, _):
            gwait(0)

            @pl.when(p > 0)
            def _():
                wwait(1)

            gstart(1, 2 * p + 1)
            wstart(0, 2 * p)
            gwait(1)
            wwait(0)
            gstart(0, 2 * p + 2)
            wstart(1, 2 * p + 1)
            return 0

        lax.fori_loop(0, (GCH - 1) // 2, pair, 0)
        gwait(0)
        wwait(1)
        wstart(0, GCH - 1)
        wwait(0)


def _gather(x, src, dst):
    f = functools.partial(
        pl.kernel,
        mesh=_sc_mesh(),
        out_type=[
            jax.ShapeDtypeStruct((E, D), jnp.float32),
            jax.ShapeDtypeStruct((E, D), jnp.float32),
        ],
        scratch_types=[
            pltpu.VMEM((E // NW,), jnp.int32),
            pltpu.VMEM((CG, D), jnp.float32),
            pltpu.VMEM((CG, D), jnp.float32),
            pltpu.SemaphoreType.DMA,
            pltpu.SemaphoreType.DMA,
            pltpu.SemaphoreType.DMA,
            pltpu.SemaphoreType.DMA,
        ],
    )(_gather_body)
    return f(x, src, dst)


def _zero_fill(buf, nrows, width):
    def zrow(r, _):
        for k in range(width // 16):
            buf[r, pl.ds(k * 16, 16)] = jnp.zeros((16,), jnp.float32)
        return 0
    lax.fori_loop(0, nrows, zrow, 0)


def _localize(idx_v, base_node, spread):
    it = lax.iota(jnp.int32, 16)
    for k in range(CS // 16):
        v = idx_v[pl.ds(k * 16, 16)]
        loc = v - base_node
        ok = (loc >= 0) & (loc < HC)
        trash = HC + ((it + k * 16) & (spread - 1))
        idx_v[pl.ds(k * 16, 16)] = jnp.where(ok, loc, trash)


SOUT = NPAD
H = 128


def _scatter_body(mi_hbm, mo_hbm, src_hbm, dst_hbm,
                  s_hbm, alo_sh, ahi_sh,
                  idx0_v, idx1_v, rlo0_v, rlo1_v, rhi0_v, rhi1_v, zb_v,
                  ls0, ls1, as0, as1):
    c = lax.axis_index("c")
    s = lax.axis_index("s")
    base_node = c * HC
    idx = (idx0_v, idx1_v)
    rlo = (rlo0_v, rlo1_v)
    rhi = (rhi0_v, rhi1_v)
    lsem = (ls0, ls1)
    asem = (as0, as1)

    _zero_fill(zb_v, 40, H)
    stripe = pl.multiple_of(s * 328, 8)
    for acc in (alo_sh, ahi_sh):
        for q in range(8):
            pltpu.sync_copy(zb_v, acc.at[pl.ds(stripe + q * 40, 40)])
        pltpu.sync_copy(zb_v.at[pl.ds(0, 8)], acc.at[pl.ds(stripe + 320, 8)])

    plsc.subcore_barrier()

    ept = E // NS
    rbase = pl.multiple_of(s * ept, 8)

    def do_array(msg_hbm, idx_hbm):
        def lstart(b, j):
            off = pl.multiple_of(j * CS, 8)
            pltpu.async_copy(idx_hbm.at[pl.ds(rbase + off, CS)],
                             idx[b], lsem[b])
            pltpu.async_copy(msg_hbm.at[pl.ds(rbase + off, CS), pl.ds(0, H)],
                             rlo[b], lsem[b])
            pltpu.async_copy(msg_hbm.at[pl.ds(rbase + off, CS), pl.ds(H, H)],
                             rhi[b], lsem[b])

        def lwait(b):
            pltpu.make_async_copy(idx_hbm.at[pl.ds(rbase, CS)],
                                  idx[b], lsem[b]).wait()
            pltpu.make_async_copy(
                msg_hbm.at[pl.ds(rbase, CS), pl.ds(0, H)],
                rlo[b], lsem[b]).wait()
            pltpu.make_async_copy(
                msg_hbm.at[pl.ds(rbase, CS), pl.ds(H, H)],
                rhi[b], lsem[b]).wait()

        def astart(b):
            pltpu.async_copy(rlo[b], alo_sh.at[idx[b]], asem[b], add=True)
            pltpu.async_copy(rhi[b], ahi_sh.at[idx[b]], asem[b], add=True)

        def await_(b):
            pltpu.make_async_copy(rlo[b], alo_sh.at[idx[b]], asem[b]).wait()
            pltpu.make_async_copy(rhi[b], ahi_sh.at[idx[b]], asem[b]).wait()

        lstart(0, 0)

        def pair(p, _):
            lwait(0)
            _localize(idx0_v, base_node, 128)

            @pl.when(p > 0)
            def _():
                await_(1)

            lstart(1, 2 * p + 1)
            astart(0)
            lwait(1)
            _localize(idx1_v, base_node, 128)
            await_(0)
            lstart(0, 2 * p + 2)
            astart(1)
            return 0

        lax.fori_loop(0, (SCH - 1) // 2, pair, 0)
        lwait(0)
        _localize(idx0_v, base_node, 128)
        await_(1)
        astart(0)
        await_(0)

    do_array(mi_hbm, dst_hbm)
    do_array(mo_hbm, src_hbm)

    plsc.subcore_barrier()

    dump = pl.multiple_of(s * 320, 8)
    obase = pl.multiple_of(c * HC + s * 320, 8)
    pltpu.sync_copy(alo_sh.at[pl.ds(dump, 320)],
                    s_hbm.at[pl.ds(obase, 320), pl.ds(0, H)])
    pltpu.sync_copy(ahi_sh.at[pl.ds(dump, 320)],
                    s_hbm.at[pl.ds(obase, 320), pl.ds(H, H)])


def _scatter(mi, mo, src, dst):
    f = functools.partial(
        pl.kernel,
        mesh=_sc_mesh(),
        out_type=jax.ShapeDtypeStruct((SOUT, D), jnp.float32),
        scratch_types=[
            pltpu.VMEM_SHARED((ACC_R, H), jnp.float32),
            pltpu.VMEM_SHARED((ACC_R, H), jnp.float32),
            pltpu.VMEM((CS,), jnp.int32),
            pltpu.VMEM((CS,), jnp.int32),
            pltpu.VMEM((CS, H), jnp.float32),
            pltpu.VMEM((CS, H), jnp.float32),
            pltpu.VMEM((CS, H), jnp.float32),
            pltpu.VMEM((CS, H), jnp.float32),
            pltpu.VMEM((40, H), jnp.float32),
            pltpu.SemaphoreType.DMA,
            pltpu.SemaphoreType.DMA,
            pltpu.SemaphoreType.DMA,
            pltpu.SemaphoreType.DMA,
        ],
    )(_scatter_body)
    return f(mi, mo, src, dst)


DACC_R = 5128


def _degree_body(src_hbm, dst_hbm, deg_hbm, dacc_sh, di_v, ones_v, zb_v, sem):
    c = lax.axis_index("c")
    s = lax.axis_index("s")
    base_node = c * HC

    _zero_fill(zb_v, 40, 128)
    stripe = pl.multiple_of(s * 320, 8)
    for q in range(8):
        pltpu.sync_copy(zb_v, dacc_sh.at[pl.ds(stripe + q * 40, 40)])

    @pl.when(s == 0)
    def _():
        pltpu.sync_copy(zb_v.at[pl.ds(0, 8)], dacc_sh.at[pl.ds(HC, 8)])

    def orow(r, _):
        for k in range(128 // 16):
            ones_v[r, pl.ds(k * 16, 16)] = jnp.ones((16,), jnp.float32)
        return 0
    lax.fori_loop(0, CS, orow, 0)

    plsc.subcore_barrier()

    ept = E // NS
    rbase = pl.multiple_of(s * ept, 8)

    def do_array(idx_hbm):
        def chunk(j, _):
            off = pl.multiple_of(j * CS, CS)
            pltpu.sync_copy(idx_hbm.at[pl.ds(rbase + off, CS)], di_v)
            _localize(di_v, base_node, 8)
            pltpu.async_copy(ones_v, dacc_sh.at[di_v], sem, add=True).wait()
            return 0

        lax.fori_loop(0, SCH, chunk, 0)

    do_array(dst_hbm)
    do_array(src_hbm)

    plsc.subcore_barrier()

    obase = pl.multiple_of(c * HC + stripe, 8)
    pltpu.sync_copy(dacc_sh.at[pl.ds(stripe, 320)],
                    deg_hbm.at[pl.ds(obase, 320)])


def _degree(src, dst):
    f = functools.partial(
        pl.kernel,
        mesh=_sc_mesh(),
        out_type=jax.ShapeDtypeStruct((NPAD, 128), jnp.float32),
        scratch_types=[
            pltpu.VMEM_SHARED((DACC_R, 128), jnp.float32),
            pltpu.VMEM((CS,), jnp.int32),
            pltpu.VMEM((CS, 128), jnp.float32),
            pltpu.VMEM((40, 128), jnp.float32),
            pltpu.SemaphoreType.DMA,
        ],
    )(_degree_body)
    return f(src, dst)


def _k1_body(xu_ref, xv_ref, fiw_ref, fow_ref,
             h1i_ref, h1o_ref, si_ref, qi_ref, so_ref, qo_ref):
    xu = xu_ref[...]
    xv = xv_ref[...]
    h1i = (jnp.dot(xu, fiw_ref[:D, :], preferred_element_type=jnp.float32)
           + jnp.dot(xv, fiw_ref[D:, :], preferred_element_type=jnp.float32))
    h1o = (jnp.dot(xv, fow_ref[:D, :], preferred_element_type=jnp.float32)
           + jnp.dot(xu, fow_ref[D:, :], preferred_element_type=jnp.float32))
    h1i_ref[...] = h1i
    h1o_ref[...] = h1o

    @pl.when(pl.program_id(0) == 0)
    def _():
        si_ref[...] = jnp.zeros_like(si_ref)
        qi_ref[...] = jnp.zeros_like(qi_ref)
        so_ref[...] = jnp.zeros_like(so_ref)
        qo_ref[...] = jnp.zeros_like(qo_ref)

    si_ref[...] += jnp.sum(h1i, axis=0, keepdims=True)
    qi_ref[...] += jnp.sum(h1i * h1i, axis=0, keepdims=True)
    so_ref[...] += jnp.sum(h1o, axis=0, keepdims=True)
    qo_ref[...] += jnp.sum(h1o * h1o, axis=0, keepdims=True)


def _affine(sum_ref, sq_ref, g_ref, be_ref, n):
    m = sum_ref[...] * (1.0 / n)
    v = sq_ref[...] * (1.0 / n) - m * m
    a = g_ref[...] * lax.rsqrt(v + EPS)
    return a, be_ref[...] - m * a


def _k2_body(h1i_ref, h1o_ref, fiw2_ref, fow2_ref,
             si_ref, qi_ref, so_ref, qo_ref,
             gi_ref, bi_ref, go_ref, bo_ref,
             h2i_ref, h2o_ref, s2i_ref, q2i_ref, s2o_ref, q2o_ref):
    ai, ci = _affine(si_ref, qi_ref, gi_ref, bi_ref, float(E))
    ao, co = _affine(so_ref, qo_ref, go_ref, bo_ref, float(E))
    x2i = jnp.maximum(h1i_ref[...] * ai + ci, 0.0)
    x2o = jnp.maximum(h1o_ref[...] * ao + co, 0.0)
    h2i = jnp.dot(x2i, fiw2_ref[...], preferred_element_type=jnp.float32)
    h2o = jnp.dot(x2o, fow2_ref[...], preferred_element_type=jnp.float32)
    h2i_ref[...] = h2i
    h2o_ref[...] = h2o

    @pl.when(pl.program_id(0) == 0)
    def _():
        s2i_ref[...] = jnp.zeros_like(s2i_ref)
        q2i_ref[...] = jnp.zeros_like(q2i_ref)
        s2o_ref[...] = jnp.zeros_like(s2o_ref)
        q2o_ref[...] = jnp.zeros_like(q2o_ref)

    s2i_ref[...] += jnp.sum(h2i, axis=0, keepdims=True)
    q2i_ref[...] += jnp.sum(h2i * h2i, axis=0, keepdims=True)
    s2o_ref[...] += jnp.sum(h2o, axis=0, keepdims=True)
    q2o_ref[...] += jnp.sum(h2o * h2o, axis=0, keepdims=True)


def _k3_body(h2i_ref, h2o_ref,
             s2i_ref, q2i_ref, s2o_ref, q2o_ref,
             gi_ref, bi_ref, go_ref, bo_ref,
             mi_ref, mo_ref):
    ai, ci = _affine(s2i_ref, q2i_ref, gi_ref, bi_ref, float(E))
    ao, co = _affine(s2o_ref, q2o_ref, go_ref, bo_ref, float(E))
    mi_ref[...] = jnp.maximum(h2i_ref[...] * ai + ci, 0.0)
    mo_ref[...] = jnp.maximum(h2o_ref[...] * ao + co, 0.0)


def _k4_body(x_ref, s_ref, deg_ref, fpw_ref, g_ref, sp_ref, qp_ref):
    dv = jnp.maximum(deg_ref[...][:, 0:1], 1.0)
    h = x_ref[...] + s_ref[...] * (1.0 / dv)
    g = jnp.dot(h, fpw_ref[...], preferred_element_type=jnp.float32)
    g_ref[...] = g

    @pl.when(pl.program_id(0) == 0)
    def _():
        sp_ref[...] = jnp.zeros_like(sp_ref)
        qp_ref[...] = jnp.zeros_like(qp_ref)

    sp_ref[...] += jnp.sum(g, axis=0, keepdims=True)
    qp_ref[...] += jnp.sum(g * g, axis=0, keepdims=True)


def _k5_body(g_ref, sp_ref, qp_ref, gg_ref, bb_ref, x_ref):
    a, c = _affine(sp_ref, qp_ref, gg_ref, bb_ref, float(N))
    x_ref[...] = jnp.maximum(g_ref[...] * a + c, 0.0)


def _row_spec(tb, d):
    return pl.BlockSpec((tb, d), lambda i: (i, 0))


def _full_spec(shape):
    return pl.BlockSpec(shape, lambda i: tuple(0 for _ in shape))


_STAT = _full_spec((1, D))


def _k1(xu, xv, fiw1, fow1):
    ge = E // TBE
    return pl.pallas_call(
        _k1_body,
        grid=(ge,),
        in_specs=[_row_spec(TBE, D), _row_spec(TBE, D),
                  _full_spec((2 * D, D)), _full_spec((2 * D, D))],
        out_specs=[_row_spec(TBE, D), _row_spec(TBE, D),
                   _STAT, _STAT, _STAT, _STAT],
        out_shape=[jax.ShapeDtypeStruct((E, D), jnp.float32),
                   jax.ShapeDtypeStruct((E, D), jnp.float32)]
                  + [jax.ShapeDtypeStruct((1, D), jnp.float32)] * 4,
    )(xu, xv, fiw1, fow1)


def _k2(h1i, h1o, fiw2, fow2, si, qi, so, qo, gi, bi, go, bo):
    ge = E // TBE
    return pl.pallas_call(
        _k2_body,
        grid=(ge,),
        in_specs=[_row_spec(TBE, D), _row_spec(TBE, D),
                  _full_spec((D, D)), _full_spec((D, D))]
                 + [_STAT] * 8,
        out_specs=[_row_spec(TBE, D), _row_spec(TBE, D),
                   _STAT, _STAT, _STAT, _STAT],
        out_shape=[jax.ShapeDtypeStruct((E, D), jnp.float32),
                   jax.ShapeDtypeStruct((E, D), jnp.float32)]
                  + [jax.ShapeDtypeStruct((1, D), jnp.float32)] * 4,
    )(h1i, h1o, fiw2, fow2, si, qi, so, qo, gi, bi, go, bo)


def _k3(h2i, h2o, s2i, q2i, s2o, q2o, gi, bi, go, bo):
    ge = E // TBE
    return pl.pallas_call(
        _k3_body,
        grid=(ge,),
        in_specs=[_row_spec(TBE, D), _row_spec(TBE, D)] + [_STAT] * 8,
        out_specs=[_row_spec(TBE, D), _row_spec(TBE, D)],
        out_shape=[jax.ShapeDtypeStruct((E, D), jnp.float32),
                   jax.ShapeDtypeStruct((E, D), jnp.float32)],
    )(h2i, h2o, s2i, q2i, s2o, q2o, gi, bi, go, bo)


def _k4(x, s, deg, fpw):
    gn = N // TBN
    return pl.pallas_call(
        _k4_body,
        grid=(gn,),
        in_specs=[_row_spec(TBN, D), _row_spec(TBN, D),
                  _row_spec(TBN, 128), _full_spec((D, D))],
        out_specs=[_row_spec(TBN, D), _STAT, _STAT],
        out_shape=[jax.ShapeDtypeStruct((N, D), jnp.float32),
                   jax.ShapeDtypeStruct((1, D), jnp.float32),
                   jax.ShapeDtypeStruct((1, D), jnp.float32)],
    )(x, s, deg, fpw)


def _k5(g, sp, qp, gg, bb):
    gn = N // TBN
    return pl.pallas_call(
        _k5_body,
        grid=(gn,),
        in_specs=[_row_spec(TBN, D)] + [_STAT] * 4,
        out_specs=_row_spec(TBN, D),
        out_shape=jax.ShapeDtypeStruct((N, D), jnp.float32),
    )(g, sp, qp, gg, bb)


def kernel(x, edge_index,
           fi_w1, fi_b1, fi_g1, fi_be1, fi_w2, fi_b2, fi_g2, fi_be2,
           fo_w1, fo_b1, fo_g1, fo_be1, fo_w2, fo_b2, fo_g2, fo_be2,
           fp_w1, fp_b1, fp_g1, fp_be1):
    del fi_b1, fi_b2, fo_b1, fo_b2, fp_b1
    r = lambda t: t.reshape(1, D)
    gi1, bi1, gi2, bi2 = r(fi_g1), r(fi_be1), r(fi_g2), r(fi_be2)
    go1, bo1, go2, bo2 = r(fo_g1), r(fo_be1), r(fo_g2), r(fo_be2)
    gp, bp = r(fp_g1), r(fp_be1)

    src, dst = edge_index[0], edge_index[1]
    deg = _degree(src, dst)[:N]
    for step in range(2):
        xu, xv = _gather(x, src, dst)
        h1i, h1o, si, qi, so, qo = _k1(xu, xv, fi_w1, fo_w1)
        h2i, h2o, s2i, q2i, s2o, q2o = _k2(
            h1i, h1o, fi_w2, fo_w2, si, qi, so, qo, gi1, bi1, go1, bo1)
        mi, mo = _k3(h2i, h2o, s2i, q2i, s2o, q2o, gi2, bi2, go2, bo2)
        s_agg = _scatter(mi, mo, src, dst)
        g, sp, qp = _k4(x, s_agg[:N], deg, fp_w1)
        x = _k5(g, sp, qp, gp, bp)
    return x


# speedup vs baseline: 1.9826x; 1.0000x over previous
import functools

import jax
import jax.numpy as jnp
from jax import lax
from jax.experimental import pallas as pl
from jax.experimental.pallas import tpu as pltpu
from jax.experimental.pallas import tpu_sc as plsc

N = 10000
E = 160000
D = 256
EPS = 1e-5

NC = 2
NS = 16
NW = NC * NS

HC = 5120
ACC_R = 5248
NPAD = 2 * HC

CG = 40
GCH = (E // NW) // CG
CS = 80
SCH = (E // NS) // CS

TBE = 2000
TBN = 2000

def _sc_mesh():
    return plsc.VectorSubcoreMesh(core_axis_name="c", subcore_axis_name="s")


def _gather_body(x_hbm, src_hbm, dst_hbm, xu_hbm, xv_hbm,
                 idx_v, rows0_v, rows1_v, gs0, gs1, ws0, ws1):
    wid = lax.axis_index("s") * NC + lax.axis_index("c")
    epw = E // NW
    base = pl.multiple_of(wid * epw, epw)
    rows = (rows0_v, rows1_v)
    gsem = (gs0, gs1)
    wsem = (ws0, ws1)

    for idx_hbm, out in ((src_hbm, xu_hbm), (dst_hbm, xv_hbm)):
        pltpu.sync_copy(idx_hbm.at[pl.ds(base, epw)], idx_v)

        def gstart(b, j):
            off = pl.multiple_of(j * CG, 8)
            pltpu.async_copy(x_hbm.at[idx_v.at[pl.ds(off, CG)]],
                             rows[b], gsem[b])

        def gwait(b):
            pltpu.make_async_copy(x_hbm.at[idx_v.at[pl.ds(0, CG)]],
                                  rows[b], gsem[b]).wait()

        def wstart(b, j):
            off = pl.multiple_of(j * CG, 8)
            pltpu.async_copy(rows[b], out.at[pl.ds(base + off, CG)], wsem[b])

        def wwait(b):
            pltpu.make_async_copy(rows[b], out.at[pl.ds(base, CG)],
                                  wsem[b]).wait()

        gstart(0, 0)

        def pair(p, _):
            gwait(0)

            @pl.when(p > 0)
            def _():
                wwait(1)

            gstart(1, 2 * p + 1)
            wstart(0, 2 * p)
            gwait(1)
            wwait(0)
            gstart(0, 2 * p + 2)
            wstart(1, 2 * p + 1)
            return 0

        lax.fori_loop(0, (GCH - 1) // 2, pair, 0)
        gwait(0)
        wwait(1)
        wstart(0, GCH - 1)
        wwait(0)


def _gather(x, src, dst):
    f = functools.partial(
        pl.kernel,
        mesh=_sc_mesh(),
        out_type=[
            jax.ShapeDtypeStruct((E, D), jnp.float32),
            jax.ShapeDtypeStruct((E, D), jnp.float32),
        ],
        scratch_types=[
            pltpu.VMEM((E // NW,), jnp.int32),
            pltpu.VMEM((CG, D), jnp.float32),
            pltpu.VMEM((CG, D), jnp.float32),
            pltpu.SemaphoreType.DMA,
            pltpu.SemaphoreType.DMA,
            pltpu.SemaphoreType.DMA,
            pltpu.SemaphoreType.DMA,
        ],
    )(_gather_body)
    return f(x, src, dst)


def _zero_fill(buf, nrows, width):
    def zrow(r, _):
        for k in range(width // 16):
            buf[r, pl.ds(k * 16, 16)] = jnp.zeros((16,), jnp.float32)
        return 0
    lax.fori_loop(0, nrows, zrow, 0)


def _localize(idx_v, base_node, spread):
    it = lax.iota(jnp.int32, 16)
    for k in range(CS // 16):
        v = idx_v[pl.ds(k * 16, 16)]
        loc = v - base_node
        ok = (loc >= 0) & (loc < HC)
        trash = HC + ((it + k * 16) & (spread - 1))
        idx_v[pl.ds(k * 16, 16)] = jnp.where(ok, loc, trash)


SOUT = NPAD
H = 128


def _scatter_body(mi_hbm, mo_hbm, src_hbm, dst_hbm,
                  s_hbm, alo_sh, ahi_sh,
                  idx0_v, idx1_v, rlo0_v, rlo1_v, rhi0_v, rhi1_v, zb_v,
                  ls0, ls1, as0, as1):
    c = lax.axis_index("c")
    s = lax.axis_index("s")
    base_node = c * HC
    idx = (idx0_v, idx1_v)
    rlo = (rlo0_v, rlo1_v)
    rhi = (rhi0_v, rhi1_v)
    lsem = (ls0, ls1)
    asem = (as0, as1)

    _zero_fill(zb_v, 40, H)
    stripe = pl.multiple_of(s * 328, 8)
    for acc in (alo_sh, ahi_sh):
        for q in range(8):
            pltpu.sync_copy(zb_v, acc.at[pl.ds(stripe + q * 40, 40)])
        pltpu.sync_copy(zb_v.at[pl.ds(0, 8)], acc.at[pl.ds(stripe + 320, 8)])

    plsc.subcore_barrier()

    ept = E // NS
    rbase = pl.multiple_of(s * ept, 8)

    def do_array(msg_hbm, idx_hbm):
        def lstart(b, j):
            off = pl.multiple_of(j * CS, 8)
            pltpu.async_copy(idx_hbm.at[pl.ds(rbase + off, CS)],
                             idx[b], lsem[b])
            pltpu.async_copy(msg_hbm.at[pl.ds(rbase + off, CS), pl.ds(0, H)],
                             rlo[b], lsem[b])
            pltpu.async_copy(msg_hbm.at[pl.ds(rbase + off, CS), pl.ds(H, H)],
                             rhi[b], lsem[b])

        def lwait(b):
            pltpu.make_async_copy(idx_hbm.at[pl.ds(rbase, CS)],
                                  idx[b], lsem[b]).wait()
            pltpu.make_async_copy(
                msg_hbm.at[pl.ds(rbase, CS), pl.ds(0, H)],
                rlo[b], lsem[b]).wait()
            pltpu.make_async_copy(
                msg_hbm.at[pl.ds(rbase, CS), pl.ds(H, H)],
                rhi[b], lsem[b]).wait()

        def astart(b):
            pltpu.async_copy(rlo[b], alo_sh.at[idx[b]], asem[b], add=True)
            pltpu.async_copy(rhi[b], ahi_sh.at[idx[b]], asem[b], add=True)

        def await_(b):
            pltpu.make_async_copy(rlo[b], alo_sh.at[idx[b]], asem[b]).wait()
            pltpu.make_async_copy(rhi[b], ahi_sh.at[idx[b]], asem[b]).wait()

        lstart(0, 0)

        def pair(p, _):
            lwait(0)
            _localize(idx0_v, base_node, 128)

            @pl.when(p > 0)
            def _():
                await_(1)

            lstart(1, 2 * p + 1)
            astart(0)
            lwait(1)
            _localize(idx1_v, base_node, 128)
            await_(0)
            lstart(0, 2 * p + 2)
            astart(1)
            return 0

        lax.fori_loop(0, (SCH - 1) // 2, pair, 0)
        lwait(0)
        _localize(idx0_v, base_node, 128)
        await_(1)
        astart(0)
        await_(0)

    do_array(mi_hbm, dst_hbm)
    do_array(mo_hbm, src_hbm)

    plsc.subcore_barrier()

    dump = pl.multiple_of(s * 320, 8)
    obase = pl.multiple_of(c * HC + s * 320, 8)
    pltpu.sync_copy(alo_sh.at[pl.ds(dump, 320)],
                    s_hbm.at[pl.ds(obase, 320), pl.ds(0, H)])
    pltpu.sync_copy(ahi_sh.at[pl.ds(dump, 320)],
                    s_hbm.at[pl.ds(obase, 320), pl.ds(H, H)])


def _scatter(mi, mo, src, dst):
    f = functools.partial(
        pl.kernel,
        mesh=_sc_mesh(),
        out_type=jax.ShapeDtypeStruct((SOUT, D), jnp.float32),
        scratch_types=[
            pltpu.VMEM_SHARED((ACC_R, H), jnp.float32),
            pltpu.VMEM_SHARED((ACC_R, H), jnp.float32),
            pltpu.VMEM((CS,), jnp.int32),
            pltpu.VMEM((CS,), jnp.int32),
            pltpu.VMEM((CS, H), jnp.float32),
            pltpu.VMEM((CS, H), jnp.float32),
            pltpu.VMEM((CS, H), jnp.float32),
            pltpu.VMEM((CS, H), jnp.float32),
            pltpu.VMEM((40, H), jnp.float32),
            pltpu.SemaphoreType.DMA,
            pltpu.SemaphoreType.DMA,
            pltpu.SemaphoreType.DMA,
            pltpu.SemaphoreType.DMA,
        ],
    )(_scatter_body)
    return f(mi, mo, src, dst)


DACC_R = 5128


def _degree_body(src_hbm, dst_hbm, deg_hbm, dacc_sh, di_v, ones_v, zb_v, sem):
    c = lax.axis_index("c")
    s = lax.axis_index("s")
    base_node = c * HC

    _zero_fill(zb_v, 40, 128)
    stripe = pl.multiple_of(s * 320, 8)
    for q in range(8):
        pltpu.sync_copy(zb_v, dacc_sh.at[pl.ds(stripe + q * 40, 40)])

    @pl.when(s == 0)
    def _():
        pltpu.sync_copy(zb_v.at[pl.ds(0, 8)], dacc_sh.at[pl.ds(HC, 8)])

    def orow(r, _):
        for k in range(128 // 16):
            ones_v[r, pl.ds(k * 16, 16)] = jnp.ones((16,), jnp.float32)
        return 0
    lax.fori_loop(0, CS, orow, 0)

    plsc.subcore_barrier()

    ept = E // NS
    rbase = pl.multiple_of(s * ept, 8)

    def do_array(idx_hbm):
        def chunk(j, _):
            off = pl.multiple_of(j * CS, CS)
            pltpu.sync_copy(idx_hbm.at[pl.ds(rbase + off, CS)], di_v)
            _localize(di_v, base_node, 8)
            pltpu.async_copy(ones_v, dacc_sh.at[di_v], sem, add=True).wait()
            return 0

        lax.fori_loop(0, SCH, chunk, 0)

    do_array(dst_hbm)
    do_array(src_hbm)

    plsc.subcore_barrier()

    obase = pl.multiple_of(c * HC + stripe, 8)
    pltpu.sync_copy(dacc_sh.at[pl.ds(stripe, 320)],
                    deg_hbm.at[pl.ds(obase, 320)])


def _degree(src, dst):
    f = functools.partial(
        pl.kernel,
        mesh=_sc_mesh(),
        out_type=jax.ShapeDtypeStruct((NPAD, 128), jnp.float32),
        scratch_types=[
            pltpu.VMEM_SHARED((DACC_R, 128), jnp.float32),
            pltpu.VMEM((CS,), jnp.int32),
            pltpu.VMEM((CS, 128), jnp.float32),
            pltpu.VMEM((40, 128), jnp.float32),
            pltpu.SemaphoreType.DMA,
        ],
    )(_degree_body)
    return f(src, dst)


def _k1_body(xu_ref, xv_ref, fiw_ref, fow_ref,
             h1i_ref, h1o_ref, si_ref, qi_ref, so_ref, qo_ref):
    xu = xu_ref[...].astype(jnp.bfloat16)
    xv = xv_ref[...].astype(jnp.bfloat16)
    h1i = (jnp.dot(xu, fiw_ref[:D, :], preferred_element_type=jnp.float32)
           + jnp.dot(xv, fiw_ref[D:, :], preferred_element_type=jnp.float32))
    h1o = (jnp.dot(xv, fow_ref[:D, :], preferred_element_type=jnp.float32)
           + jnp.dot(xu, fow_ref[D:, :], preferred_element_type=jnp.float32))
    h1i_ref[...] = h1i
    h1o_ref[...] = h1o

    @pl.when(pl.program_id(0) == 0)
    def _():
        si_ref[...] = jnp.zeros_like(si_ref)
        qi_ref[...] = jnp.zeros_like(qi_ref)
        so_ref[...] = jnp.zeros_like(so_ref)
        qo_ref[...] = jnp.zeros_like(qo_ref)

    si_ref[...] += jnp.sum(h1i, axis=0, keepdims=True)
    qi_ref[...] += jnp.sum(h1i * h1i, axis=0, keepdims=True)
    so_ref[...] += jnp.sum(h1o, axis=0, keepdims=True)
    qo_ref[...] += jnp.sum(h1o * h1o, axis=0, keepdims=True)


def _affine(sum_ref, sq_ref, g_ref, be_ref, n):
    m = sum_ref[...] * (1.0 / n)
    v = sq_ref[...] * (1.0 / n) - m * m
    a = g_ref[...] * lax.rsqrt(v + EPS)
    return a, be_ref[...] - m * a


def _k2_body(h1i_ref, h1o_ref, fiw2_ref, fow2_ref,
             si_ref, qi_ref, so_ref, qo_ref,
             gi_ref, bi_ref, go_ref, bo_ref,
             h2i_ref, h2o_ref, s2i_ref, q2i_ref, s2o_ref, q2o_ref):
    ai, ci = _affine(si_ref, qi_ref, gi_ref, bi_ref, float(E))
    ao, co = _affine(so_ref, qo_ref, go_ref, bo_ref, float(E))
    x2i = jnp.maximum(h1i_ref[...] * ai + ci, 0.0).astype(jnp.bfloat16)
    x2o = jnp.maximum(h1o_ref[...] * ao + co, 0.0).astype(jnp.bfloat16)
    h2i = jnp.dot(x2i, fiw2_ref[...], preferred_element_type=jnp.float32)
    h2o = jnp.dot(x2o, fow2_ref[...], preferred_element_type=jnp.float32)
    h2i_ref[...] = h2i
    h2o_ref[...] = h2o

    @pl.when(pl.program_id(0) == 0)
    def _():
        s2i_ref[...] = jnp.zeros_like(s2i_ref)
        q2i_ref[...] = jnp.zeros_like(q2i_ref)
        s2o_ref[...] = jnp.zeros_like(s2o_ref)
        q2o_ref[...] = jnp.zeros_like(q2o_ref)

    s2i_ref[...] += jnp.sum(h2i, axis=0, keepdims=True)
    q2i_ref[...] += jnp.sum(h2i * h2i, axis=0, keepdims=True)
    s2o_ref[...] += jnp.sum(h2o, axis=0, keepdims=True)
    q2o_ref[...] += jnp.sum(h2o * h2o, axis=0, keepdims=True)


def _k3_body(h2i_ref, h2o_ref,
             s2i_ref, q2i_ref, s2o_ref, q2o_ref,
             gi_ref, bi_ref, go_ref, bo_ref,
             mi_ref, mo_ref):
    ai, ci = _affine(s2i_ref, q2i_ref, gi_ref, bi_ref, float(E))
    ao, co = _affine(s2o_ref, q2o_ref, go_ref, bo_ref, float(E))
    mi_ref[...] = jnp.maximum(h2i_ref[...] * ai + ci, 0.0)
    mo_ref[...] = jnp.maximum(h2o_ref[...] * ao + co, 0.0)


def _k4_body(x_ref, s_ref, deg_ref, fpw_ref, g_ref, sp_ref, qp_ref):
    dv = jnp.maximum(deg_ref[...][:, 0:1], 1.0)
    h = (x_ref[...] + s_ref[...] * (1.0 / dv)).astype(jnp.bfloat16)
    g = jnp.dot(h, fpw_ref[...], preferred_element_type=jnp.float32)
    g_ref[...] = g

    @pl.when(pl.program_id(0) == 0)
    def _():
        sp_ref[...] = jnp.zeros_like(sp_ref)
        qp_ref[...] = jnp.zeros_like(qp_ref)

    sp_ref[...] += jnp.sum(g, axis=0, keepdims=True)
    qp_ref[...] += jnp.sum(g * g, axis=0, keepdims=True)


def _k5_body(g_ref, sp_ref, qp_ref, gg_ref, bb_ref, x_ref):
    a, c = _affine(sp_ref, qp_ref, gg_ref, bb_ref, float(N))
    x_ref[...] = jnp.maximum(g_ref[...] * a + c, 0.0)


def _row_spec(tb, d):
    return pl.BlockSpec((tb, d), lambda i: (i, 0))


def _full_spec(shape):
    return pl.BlockSpec(shape, lambda i: tuple(0 for _ in shape))


_STAT = _full_spec((1, D))


def _k1(xu, xv, fiw1, fow1):
    ge = E // TBE
    return pl.pallas_call(
        _k1_body,
        grid=(ge,),
        in_specs=[_row_spec(TBE, D), _row_spec(TBE, D),
                  _full_spec((2 * D, D)), _full_spec((2 * D, D))],
        out_specs=[_row_spec(TBE, D), _row_spec(TBE, D),
                   _STAT, _STAT, _STAT, _STAT],
        out_shape=[jax.ShapeDtypeStruct((E, D), jnp.float32),
                   jax.ShapeDtypeStruct((E, D), jnp.float32)]
                  + [jax.ShapeDtypeStruct((1, D), jnp.float32)] * 4,
    )(xu, xv, fiw1, fow1)


def _k2(h1i, h1o, fiw2, fow2, si, qi, so, qo, gi, bi, go, bo):
    ge = E // TBE
    return pl.pallas_call(
        _k2_body,
        grid=(ge,),
        in_specs=[_row_spec(TBE, D), _row_spec(TBE, D),
                  _full_spec((D, D)), _full_spec((D, D))]
                 + [_STAT] * 8,
        out_specs=[_row_spec(TBE, D), _row_spec(TBE, D),
                   _STAT, _STAT, _STAT, _STAT],
        out_shape=[jax.ShapeDtypeStruct((E, D), jnp.float32),
                   jax.ShapeDtypeStruct((E, D), jnp.float32)]
                  + [jax.ShapeDtypeStruct((1, D), jnp.float32)] * 4,
    )(h1i, h1o, fiw2, fow2, si, qi, so, qo, gi, bi, go, bo)


def _k3(h2i, h2o, s2i, q2i, s2o, q2o, gi, bi, go, bo):
    ge = E // TBE
    return pl.pallas_call(
        _k3_body,
        grid=(ge,),
        in_specs=[_row_spec(TBE, D), _row_spec(TBE, D)] + [_STAT] * 8,
        out_specs=[_row_spec(TBE, D), _row_spec(TBE, D)],
        out_shape=[jax.ShapeDtypeStruct((E, D), jnp.float32),
                   jax.ShapeDtypeStruct((E, D), jnp.float32)],
    )(h2i, h2o, s2i, q2i, s2o, q2o, gi, bi, go, bo)


def _k4(x, s, deg, fpw):
    gn = N // TBN
    return pl.pallas_call(
        _k4_body,
        grid=(gn,),
        in_specs=[_row_spec(TBN, D), _row_spec(TBN, D),
                  _row_spec(TBN, 128), _full_spec((D, D))],
        out_specs=[_row_spec(TBN, D), _STAT, _STAT],
        out_shape=[jax.ShapeDtypeStruct((N, D), jnp.float32),
                   jax.ShapeDtypeStruct((1, D), jnp.float32),
                   jax.ShapeDtypeStruct((1, D), jnp.float32)],
    )(x, s, deg, fpw)


def _k5(g, sp, qp, gg, bb):
    gn = N // TBN
    return pl.pallas_call(
        _k5_body,
        grid=(gn,),
        in_specs=[_row_spec(TBN, D)] + [_STAT] * 4,
        out_specs=_row_spec(TBN, D),
        out_shape=jax.ShapeDtypeStruct((N, D), jnp.float32),
    )(g, sp, qp, gg, bb)


def kernel(x, edge_index,
           fi_w1, fi_b1, fi_g1, fi_be1, fi_w2, fi_b2, fi_g2, fi_be2,
           fo_w1, fo_b1, fo_g1, fo_be1, fo_w2, fo_b2, fo_g2, fo_be2,
           fp_w1, fp_b1, fp_g1, fp_be1):
    del fi_b1, fi_b2, fo_b1, fo_b2, fp_b1
    r = lambda t: t.reshape(1, D)
    gi1, bi1, gi2, bi2 = r(fi_g1), r(fi_be1), r(fi_g2), r(fi_be2)
    go1, bo1, go2, bo2 = r(fo_g1), r(fo_be1), r(fo_g2), r(fo_be2)
    gp, bp = r(fp_g1), r(fp_be1)
    fi_w1 = fi_w1.astype(jnp.bfloat16)
    fo_w1 = fo_w1.astype(jnp.bfloat16)
    fi_w2 = fi_w2.astype(jnp.bfloat16)
    fo_w2 = fo_w2.astype(jnp.bfloat16)
    fp_w1 = fp_w1.astype(jnp.bfloat16)

    src, dst = edge_index[0], edge_index[1]
    deg = _degree(src, dst)[:N]
    for step in range(2):
        xu, xv = _gather(x, src, dst)
        h1i, h1o, si, qi, so, qo = _k1(xu, xv, fi_w1, fo_w1)
        h2i, h2o, s2i, q2i, s2o, q2o = _k2(
            h1i, h1o, fi_w2, fo_w2, si, qi, so, qo, gi1, bi1, go1, bo1)
        mi, mo = _k3(h2i, h2o, s2i, q2i, s2o, q2o, gi2, bi2, go2, bo2)
        s_agg = _scatter(mi, mo, src, dst)
        g, sp, qp = _k4(x, s_agg[:N], deg, fp_w1)
        x = _k5(g, sp, qp, gp, bp)
    return x


# speedup vs baseline: 2.2031x; 1.1112x over previous
import functools

import jax
import jax.numpy as jnp
from jax import lax
from jax.experimental import pallas as pl
from jax.experimental.pallas import tpu as pltpu
from jax.experimental.pallas import tpu_sc as plsc

N = 10000
E = 160000
D = 256
EPS = 1e-5

NC = 2
NS = 16
NW = NC * NS

HC = 5120
ACC_R = 5248
NPAD = 2 * HC

CG = 40
GCH = (E // NW) // CG
CS = 80
SCH = (E // NS) // CS

TBE = 2000
TBN = 2000

def _sc_mesh():
    return plsc.VectorSubcoreMesh(core_axis_name="c", subcore_axis_name="s")


def _gather_body(x_hbm, src_hbm, dst_hbm, xu_hbm, xv_hbm,
                 idx_v, rows0_v, rows1_v, gs0, gs1, ws0, ws1):
    wid = lax.axis_index("s") * NC + lax.axis_index("c")
    epw = E // NW
    base = pl.multiple_of(wid * epw, epw)
    rows = (rows0_v, rows1_v)
    gsem = (gs0, gs1)
    wsem = (ws0, ws1)

    for idx_hbm, out in ((src_hbm, xu_hbm), (dst_hbm, xv_hbm)):
        pltpu.sync_copy(idx_hbm.at[pl.ds(base, epw)], idx_v)

        def gstart(b, j):
            off = pl.multiple_of(j * CG, 8)
            pltpu.async_copy(x_hbm.at[idx_v.at[pl.ds(off, CG)]],
                             rows[b], gsem[b])

        def gwait(b):
            pltpu.make_async_copy(x_hbm.at[idx_v.at[pl.ds(0, CG)]],
                                  rows[b], gsem[b]).wait()

        def wstart(b, j):
            off = pl.multiple_of(j * CG, 8)
            pltpu.async_copy(rows[b], out.at[pl.ds(base + off, CG)], wsem[b])

        def wwait(b):
            pltpu.make_async_copy(rows[b], out.at[pl.ds(base, CG)],
                                  wsem[b]).wait()

        gstart(0, 0)

        def pair(p, _):
            gwait(0)

            @pl.when(p > 0)
            def _():
                wwait(1)

            gstart(1, 2 * p + 1)
            wstart(0, 2 * p)
            gwait(1)
            wwait(0)
            gstart(0, 2 * p + 2)
            wstart(1, 2 * p + 1)
            return 0

        lax.fori_loop(0, (GCH - 1) // 2, pair, 0)
        gwait(0)
        wwait(1)
        wstart(0, GCH - 1)
        wwait(0)


def _gather(x, src, dst):
    f = functools.partial(
        pl.kernel,
        mesh=_sc_mesh(),
        out_type=[
            jax.ShapeDtypeStruct((E, D), jnp.float32),
            jax.ShapeDtypeStruct((E, D), jnp.float32),
        ],
        scratch_types=[
            pltpu.VMEM((E // NW,), jnp.int32),
            pltpu.VMEM((CG, D), jnp.float32),
            pltpu.VMEM((CG, D), jnp.float32),
            pltpu.SemaphoreType.DMA,
            pltpu.SemaphoreType.DMA,
            pltpu.SemaphoreType.DMA,
            pltpu.SemaphoreType.DMA,
        ],
    )(_gather_body)
    return f(x, src, dst)


def _zero_fill(buf, nrows, width):
    def zrow(r, _):
        for k in range(width // 16):
            buf[r, pl.ds(k * 16, 16)] = jnp.zeros((16,), jnp.float32)
        return 0
    lax.fori_loop(0, nrows, zrow, 0)


def _localize(idx_v, base_node, spread):
    it = lax.iota(jnp.int32, 16)
    for k in range(CS // 16):
        v = idx_v[pl.ds(k * 16, 16)]
        loc = v - base_node
        ok = (loc >= 0) & (loc < HC)
        trash = HC + ((it + k * 16) & (spread - 1))
        idx_v[pl.ds(k * 16, 16)] = jnp.where(ok, loc, trash)


SOUT = NPAD
H = 128


def _scatter_body(mi_hbm, mo_hbm, src_hbm, dst_hbm,
                  s_hbm, alo_sh, ahi_sh,
                  idx0_v, idx1_v, rlo0_v, rlo1_v, rhi0_v, rhi1_v, zb_v,
                  ls0, ls1, as0, as1):
    c = lax.axis_index("c")
    s = lax.axis_index("s")
    base_node = c * HC
    idx = (idx0_v, idx1_v)
    rlo = (rlo0_v, rlo1_v)
    rhi = (rhi0_v, rhi1_v)
    lsem = (ls0, ls1)
    asem = (as0, as1)

    _zero_fill(zb_v, 40, H)
    stripe = pl.multiple_of(s * 328, 8)
    for acc in (alo_sh, ahi_sh):
        for q in range(8):
            pltpu.sync_copy(zb_v, acc.at[pl.ds(stripe + q * 40, 40)])
        pltpu.sync_copy(zb_v.at[pl.ds(0, 8)], acc.at[pl.ds(stripe + 320, 8)])

    plsc.subcore_barrier()

    ept = E // NS
    rbase = pl.multiple_of(s * ept, 8)

    def do_array(msg_hbm, idx_hbm):
        def lstart(b, j):
            off = pl.multiple_of(j * CS, 8)
            pltpu.async_copy(idx_hbm.at[pl.ds(rbase + off, CS)],
                             idx[b], lsem[b])
            pltpu.async_copy(msg_hbm.at[pl.ds(rbase + off, CS), pl.ds(0, H)],
                             rlo[b], lsem[b])
            pltpu.async_copy(msg_hbm.at[pl.ds(rbase + off, CS), pl.ds(H, H)],
                             rhi[b], lsem[b])

        def lwait(b):
            pltpu.make_async_copy(idx_hbm.at[pl.ds(rbase, CS)],
                                  idx[b], lsem[b]).wait()
            pltpu.make_async_copy(
                msg_hbm.at[pl.ds(rbase, CS), pl.ds(0, H)],
                rlo[b], lsem[b]).wait()
            pltpu.make_async_copy(
                msg_hbm.at[pl.ds(rbase, CS), pl.ds(H, H)],
                rhi[b], lsem[b]).wait()

        def astart(b):
            pltpu.async_copy(rlo[b], alo_sh.at[idx[b]], asem[b], add=True)
            pltpu.async_copy(rhi[b], ahi_sh.at[idx[b]], asem[b], add=True)

        def await_(b):
            pltpu.make_async_copy(rlo[b], alo_sh.at[idx[b]], asem[b]).wait()
            pltpu.make_async_copy(rhi[b], ahi_sh.at[idx[b]], asem[b]).wait()

        lstart(0, 0)

        def pair(p, _):
            lwait(0)
            _localize(idx0_v, base_node, 128)

            @pl.when(p > 0)
            def _():
                await_(1)

            lstart(1, 2 * p + 1)
            astart(0)
            lwait(1)
            _localize(idx1_v, base_node, 128)
            await_(0)
            lstart(0, 2 * p + 2)
            astart(1)
            return 0

        lax.fori_loop(0, (SCH - 1) // 2, pair, 0)
        lwait(0)
        _localize(idx0_v, base_node, 128)
        await_(1)
        astart(0)
        await_(0)

    do_array(mi_hbm, dst_hbm)
    do_array(mo_hbm, src_hbm)

    plsc.subcore_barrier()

    dump = pl.multiple_of(s * 320, 8)
    obase = pl.multiple_of(c * HC + s * 320, 8)
    pltpu.sync_copy(alo_sh.at[pl.ds(dump, 320)],
                    s_hbm.at[pl.ds(obase, 320), pl.ds(0, H)])
    pltpu.sync_copy(ahi_sh.at[pl.ds(dump, 320)],
                    s_hbm.at[pl.ds(obase, 320), pl.ds(H, H)])


def _scatter(mi, mo, src, dst):
    f = functools.partial(
        pl.kernel,
        mesh=_sc_mesh(),
        out_type=jax.ShapeDtypeStruct((SOUT, D), jnp.float32),
        scratch_types=[
            pltpu.VMEM_SHARED((ACC_R, H), jnp.float32),
            pltpu.VMEM_SHARED((ACC_R, H), jnp.float32),
            pltpu.VMEM((CS,), jnp.int32),
            pltpu.VMEM((CS,), jnp.int32),
            pltpu.VMEM((CS, H), jnp.float32),
            pltpu.VMEM((CS, H), jnp.float32),
            pltpu.VMEM((CS, H), jnp.float32),
            pltpu.VMEM((CS, H), jnp.float32),
            pltpu.VMEM((40, H), jnp.float32),
            pltpu.SemaphoreType.DMA,
            pltpu.SemaphoreType.DMA,
            pltpu.SemaphoreType.DMA,
            pltpu.SemaphoreType.DMA,
        ],
    )(_scatter_body)
    return f(mi, mo, src, dst)


DACC_R = 5128


def _degree_body(src_hbm, dst_hbm, deg_hbm, dacc_sh, di_v, ones_v, zb_v, sem):
    c = lax.axis_index("c")
    s = lax.axis_index("s")
    base_node = c * HC

    _zero_fill(zb_v, 40, 128)
    stripe = pl.multiple_of(s * 320, 8)
    for q in range(8):
        pltpu.sync_copy(zb_v, dacc_sh.at[pl.ds(stripe + q * 40, 40)])

    @pl.when(s == 0)
    def _():
        pltpu.sync_copy(zb_v.at[pl.ds(0, 8)], dacc_sh.at[pl.ds(HC, 8)])

    def orow(r, _):
        for k in range(128 // 16):
            ones_v[r, pl.ds(k * 16, 16)] = jnp.ones((16,), jnp.float32)
        return 0
    lax.fori_loop(0, CS, orow, 0)

    plsc.subcore_barrier()

    ept = E // NS
    rbase = pl.multiple_of(s * ept, 8)

    def do_array(idx_hbm):
        def chunk(j, _):
            off = pl.multiple_of(j * CS, CS)
            pltpu.sync_copy(idx_hbm.at[pl.ds(rbase + off, CS)], di_v)
            _localize(di_v, base_node, 8)
            pltpu.async_copy(ones_v, dacc_sh.at[di_v], sem, add=True).wait()
            return 0

        lax.fori_loop(0, SCH, chunk, 0)

    do_array(dst_hbm)
    do_array(src_hbm)

    plsc.subcore_barrier()

    obase = pl.multiple_of(c * HC + stripe, 8)
    pltpu.sync_copy(dacc_sh.at[pl.ds(stripe, 320)],
                    deg_hbm.at[pl.ds(obase, 320)])


def _degree(src, dst):
    f = functools.partial(
        pl.kernel,
        mesh=_sc_mesh(),
        out_type=jax.ShapeDtypeStruct((NPAD, 128), jnp.float32),
        scratch_types=[
            pltpu.VMEM_SHARED((DACC_R, 128), jnp.float32),
            pltpu.VMEM((CS,), jnp.int32),
            pltpu.VMEM((CS, 128), jnp.float32),
            pltpu.VMEM((40, 128), jnp.float32),
            pltpu.SemaphoreType.DMA,
        ],
    )(_degree_body)
    return f(src, dst)


def _k1_body(xu_ref, xv_ref, fiw_ref, fow_ref,
             h1i_ref, h1o_ref, si_ref, qi_ref, so_ref, qo_ref):
    xu = xu_ref[...].astype(jnp.bfloat16)
    xv = xv_ref[...].astype(jnp.bfloat16)
    h1i = (jnp.dot(xu, fiw_ref[:D, :], preferred_element_type=jnp.float32)
           + jnp.dot(xv, fiw_ref[D:, :], preferred_element_type=jnp.float32))
    h1o = (jnp.dot(xv, fow_ref[:D, :], preferred_element_type=jnp.float32)
           + jnp.dot(xu, fow_ref[D:, :], preferred_element_type=jnp.float32))
    h1i_ref[...] = h1i.astype(jnp.bfloat16)
    h1o_ref[...] = h1o.astype(jnp.bfloat16)

    @pl.when(pl.program_id(0) == 0)
    def _():
        si_ref[...] = jnp.zeros_like(si_ref)
        qi_ref[...] = jnp.zeros_like(qi_ref)
        so_ref[...] = jnp.zeros_like(so_ref)
        qo_ref[...] = jnp.zeros_like(qo_ref)

    si_ref[...] += jnp.sum(h1i, axis=0, keepdims=True)
    qi_ref[...] += jnp.sum(h1i * h1i, axis=0, keepdims=True)
    so_ref[...] += jnp.sum(h1o, axis=0, keepdims=True)
    qo_ref[...] += jnp.sum(h1o * h1o, axis=0, keepdims=True)


def _affine(sum_ref, sq_ref, g_ref, be_ref, n):
    m = sum_ref[...] * (1.0 / n)
    v = sq_ref[...] * (1.0 / n) - m * m
    a = g_ref[...] * lax.rsqrt(v + EPS)
    return a, be_ref[...] - m * a


def _k2_body(h1i_ref, h1o_ref, fiw2_ref, fow2_ref,
             si_ref, qi_ref, so_ref, qo_ref,
             gi_ref, bi_ref, go_ref, bo_ref,
             h2i_ref, h2o_ref, s2i_ref, q2i_ref, s2o_ref, q2o_ref):
    ai, ci = _affine(si_ref, qi_ref, gi_ref, bi_ref, float(E))
    ao, co = _affine(so_ref, qo_ref, go_ref, bo_ref, float(E))
    x2i = jnp.maximum(h1i_ref[...].astype(jnp.float32) * ai + ci,
                      0.0).astype(jnp.bfloat16)
    x2o = jnp.maximum(h1o_ref[...].astype(jnp.float32) * ao + co,
                      0.0).astype(jnp.bfloat16)
    h2i = jnp.dot(x2i, fiw2_ref[...], preferred_element_type=jnp.float32)
    h2o = jnp.dot(x2o, fow2_ref[...], preferred_element_type=jnp.float32)
    h2i_ref[...] = h2i.astype(jnp.bfloat16)
    h2o_ref[...] = h2o.astype(jnp.bfloat16)

    @pl.when(pl.program_id(0) == 0)
    def _():
        s2i_ref[...] = jnp.zeros_like(s2i_ref)
        q2i_ref[...] = jnp.zeros_like(q2i_ref)
        s2o_ref[...] = jnp.zeros_like(s2o_ref)
        q2o_ref[...] = jnp.zeros_like(q2o_ref)

    s2i_ref[...] += jnp.sum(h2i, axis=0, keepdims=True)
    q2i_ref[...] += jnp.sum(h2i * h2i, axis=0, keepdims=True)
    s2o_ref[...] += jnp.sum(h2o, axis=0, keepdims=True)
    q2o_ref[...] += jnp.sum(h2o * h2o, axis=0, keepdims=True)


def _k3_body(h2i_ref, h2o_ref,
             s2i_ref, q2i_ref, s2o_ref, q2o_ref,
             gi_ref, bi_ref, go_ref, bo_ref,
             mi_ref, mo_ref):
    ai, ci = _affine(s2i_ref, q2i_ref, gi_ref, bi_ref, float(E))
    ao, co = _affine(s2o_ref, q2o_ref, go_ref, bo_ref, float(E))
    mi_ref[...] = jnp.maximum(h2i_ref[...].astype(jnp.float32) * ai + ci, 0.0)
    mo_ref[...] = jnp.maximum(h2o_ref[...].astype(jnp.float32) * ao + co, 0.0)


def _k4_body(x_ref, s_ref, deg_ref, fpw_ref, g_ref, sp_ref, qp_ref):
    dv = jnp.maximum(deg_ref[...][:, 0:1], 1.0)
    h = (x_ref[...] + s_ref[...] * (1.0 / dv)).astype(jnp.bfloat16)
    g = jnp.dot(h, fpw_ref[...], preferred_element_type=jnp.float32)
    g_ref[...] = g

    @pl.when(pl.program_id(0) == 0)
    def _():
        sp_ref[...] = jnp.zeros_like(sp_ref)
        qp_ref[...] = jnp.zeros_like(qp_ref)

    sp_ref[...] += jnp.sum(g, axis=0, keepdims=True)
    qp_ref[...] += jnp.sum(g * g, axis=0, keepdims=True)


def _k5_body(g_ref, sp_ref, qp_ref, gg_ref, bb_ref, x_ref):
    a, c = _affine(sp_ref, qp_ref, gg_ref, bb_ref, float(N))
    x_ref[...] = jnp.maximum(g_ref[...] * a + c, 0.0)


def _row_spec(tb, d):
    return pl.BlockSpec((tb, d), lambda i: (i, 0))


def _full_spec(shape):
    return pl.BlockSpec(shape, lambda i: tuple(0 for _ in shape))


_STAT = _full_spec((1, D))


def _k1(xu, xv, fiw1, fow1):
    ge = E // TBE
    return pl.pallas_call(
        _k1_body,
        grid=(ge,),
        in_specs=[_row_spec(TBE, D), _row_spec(TBE, D),
                  _full_spec((2 * D, D)), _full_spec((2 * D, D))],
        out_specs=[_row_spec(TBE, D), _row_spec(TBE, D),
                   _STAT, _STAT, _STAT, _STAT],
        out_shape=[jax.ShapeDtypeStruct((E, D), jnp.bfloat16),
                   jax.ShapeDtypeStruct((E, D), jnp.bfloat16)]
                  + [jax.ShapeDtypeStruct((1, D), jnp.float32)] * 4,
    )(xu, xv, fiw1, fow1)


def _k2(h1i, h1o, fiw2, fow2, si, qi, so, qo, gi, bi, go, bo):
    ge = E // TBE
    return pl.pallas_call(
        _k2_body,
        grid=(ge,),
        in_specs=[_row_spec(TBE, D), _row_spec(TBE, D),
                  _full_spec((D, D)), _full_spec((D, D))]
                 + [_STAT] * 8,
        out_specs=[_row_spec(TBE, D), _row_spec(TBE, D),
                   _STAT, _STAT, _STAT, _STAT],
        out_shape=[jax.ShapeDtypeStruct((E, D), jnp.bfloat16),
                   jax.ShapeDtypeStruct((E, D), jnp.bfloat16)]
                  + [jax.ShapeDtypeStruct((1, D), jnp.float32)] * 4,
    )(h1i, h1o, fiw2, fow2, si, qi, so, qo, gi, bi, go, bo)


def _k3(h2i, h2o, s2i, q2i, s2o, q2o, gi, bi, go, bo):
    ge = E // TBE
    return pl.pallas_call(
        _k3_body,
        grid=(ge,),
        in_specs=[_row_spec(TBE, D), _row_spec(TBE, D)] + [_STAT] * 8,
        out_specs=[_row_spec(TBE, D), _row_spec(TBE, D)],
        out_shape=[jax.ShapeDtypeStruct((E, D), jnp.float32),
                   jax.ShapeDtypeStruct((E, D), jnp.float32)],
    )(h2i, h2o, s2i, q2i, s2o, q2o, gi, bi, go, bo)


def _k4(x, s, deg, fpw):
    gn = N // TBN
    return pl.pallas_call(
        _k4_body,
        grid=(gn,),
        in_specs=[_row_spec(TBN, D), _row_spec(TBN, D),
                  _row_spec(TBN, 128), _full_spec((D, D))],
        out_specs=[_row_spec(TBN, D), _STAT, _STAT],
        out_shape=[jax.ShapeDtypeStruct((N, D), jnp.float32),
                   jax.ShapeDtypeStruct((1, D), jnp.float32),
                   jax.ShapeDtypeStruct((1, D), jnp.float32)],
    )(x, s, deg, fpw)


def _k5(g, sp, qp, gg, bb):
    gn = N // TBN
    return pl.pallas_call(
        _k5_body,
        grid=(gn,),
        in_specs=[_row_spec(TBN, D)] + [_STAT] * 4,
        out_specs=_row_spec(TBN, D),
        out_shape=jax.ShapeDtypeStruct((N, D), jnp.float32),
    )(g, sp, qp, gg, bb)


def kernel(x, edge_index,
           fi_w1, fi_b1, fi_g1, fi_be1, fi_w2, fi_b2, fi_g2, fi_be2,
           fo_w1, fo_b1, fo_g1, fo_be1, fo_w2, fo_b2, fo_g2, fo_be2,
           fp_w1, fp_b1, fp_g1, fp_be1):
    del fi_b1, fi_b2, fo_b1, fo_b2, fp_b1
    r = lambda t: t.reshape(1, D)
    gi1, bi1, gi2, bi2 = r(fi_g1), r(fi_be1), r(fi_g2), r(fi_be2)
    go1, bo1, go2, bo2 = r(fo_g1), r(fo_be1), r(fo_g2), r(fo_be2)
    gp, bp = r(fp_g1), r(fp_be1)
    fi_w1 = fi_w1.astype(jnp.bfloat16)
    fo_w1 = fo_w1.astype(jnp.bfloat16)
    fi_w2 = fi_w2.astype(jnp.bfloat16)
    fo_w2 = fo_w2.astype(jnp.bfloat16)
    fp_w1 = fp_w1.astype(jnp.bfloat16)

    src, dst = edge_index[0], edge_index[1]
    deg = _degree(src, dst)[:N]
    for step in range(2):
        xu, xv = _gather(x, src, dst)
        h1i, h1o, si, qi, so, qo = _k1(xu, xv, fi_w1, fo_w1)
        h2i, h2o, s2i, q2i, s2o, q2o = _k2(
            h1i, h1o, fi_w2, fo_w2, si, qi, so, qo, gi1, bi1, go1, bo1)
        mi, mo = _k3(h2i, h2o, s2i, q2i, s2o, q2o, gi2, bi2, go2, bo2)
        s_agg = _scatter(mi, mo, src, dst)
        g, sp, qp = _k4(x, s_agg[:N], deg, fp_w1)
        x = _k5(g, sp, qp, gp, bp)
    return x


# speedup vs baseline: 2.2035x; 1.0002x over previous
import functools

import jax
import jax.numpy as jnp
from jax import lax
from jax.experimental import pallas as pl
from jax.experimental.pallas import tpu as pltpu
from jax.experimental.pallas import tpu_sc as plsc

N = 10000
E = 160000
D = 256
EPS = 1e-5

NC = 2
NS = 16
NW = NC * NS

HC = 5120
ACC_R = 5248
NPAD = 2 * HC

CG = 40
GCH = (E // NW) // CG
CS = 80
SCH = (E // NS) // CS

TBE = 2000
TBN = 2000

def _sc_mesh():
    return plsc.VectorSubcoreMesh(core_axis_name="c", subcore_axis_name="s")


def _gather_body(x_hbm, src_hbm, dst_hbm, xu_hbm, xv_hbm,
                 idx_v, rows0_v, rows1_v, gs0, gs1, ws0, ws1):
    wid = lax.axis_index("s") * NC + lax.axis_index("c")
    epw = E // NW
    base = pl.multiple_of(wid * epw, epw)
    rows = (rows0_v, rows1_v)
    gsem = (gs0, gs1)
    wsem = (ws0, ws1)

    for idx_hbm, out in ((src_hbm, xu_hbm), (dst_hbm, xv_hbm)):
        pltpu.sync_copy(idx_hbm.at[pl.ds(base, epw)], idx_v)

        def gstart(b, j):
            off = pl.multiple_of(j * CG, 8)
            pltpu.async_copy(x_hbm.at[idx_v.at[pl.ds(off, CG)]],
                             rows[b], gsem[b])

        def gwait(b):
            pltpu.make_async_copy(x_hbm.at[idx_v.at[pl.ds(0, CG)]],
                                  rows[b], gsem[b]).wait()

        def wstart(b, j):
            off = pl.multiple_of(j * CG, 8)
            pltpu.async_copy(rows[b], out.at[pl.ds(base + off, CG)], wsem[b])

        def wwait(b):
            pltpu.make_async_copy(rows[b], out.at[pl.ds(base, CG)],
                                  wsem[b]).wait()

        gstart(0, 0)

        def pair(p, _):
            gwait(0)

            @pl.when(p > 0)
            def _():
                wwait(1)

            gstart(1, 2 * p + 1)
            wstart(0, 2 * p)
            gwait(1)
            wwait(0)
            gstart(0, 2 * p + 2)
            wstart(1, 2 * p + 1)
            return 0

        lax.fori_loop(0, (GCH - 1) // 2, pair, 0)
        gwait(0)
        wwait(1)
        wstart(0, GCH - 1)
        wwait(0)


def _gather(x, src, dst):
    f = functools.partial(
        pl.kernel,
        mesh=_sc_mesh(),
        out_type=[
            jax.ShapeDtypeStruct((E, D), jnp.float32),
            jax.ShapeDtypeStruct((E, D), jnp.float32),
        ],
        scratch_types=[
            pltpu.VMEM((E // NW,), jnp.int32),
            pltpu.VMEM((CG, D), jnp.float32),
            pltpu.VMEM((CG, D), jnp.float32),
            pltpu.SemaphoreType.DMA,
            pltpu.SemaphoreType.DMA,
            pltpu.SemaphoreType.DMA,
            pltpu.SemaphoreType.DMA,
        ],
    )(_gather_body)
    return f(x, src, dst)


def _zero_fill(buf, nrows, width):
    def zrow(r, _):
        for k in range(width // 16):
            buf[r, pl.ds(k * 16, 16)] = jnp.zeros((16,), jnp.float32)
        return 0
    lax.fori_loop(0, nrows, zrow, 0)


def _localize(idx_v, base_node, spread):
    it = lax.iota(jnp.int32, 16)
    for k in range(CS // 16):
        v = idx_v[pl.ds(k * 16, 16)]
        loc = v - base_node
        ok = (loc >= 0) & (loc < HC)
        trash = HC + ((it + k * 16) & (spread - 1))
        idx_v[pl.ds(k * 16, 16)] = jnp.where(ok, loc, trash)


SOUT = NPAD
H = 128


def _scatter_body(mi_hbm, mo_hbm, src_hbm, dst_hbm,
                  s_hbm, alo_sh, ahi_sh,
                  idx0_v, idx1_v, rlo0_v, rlo1_v, rhi0_v, rhi1_v, zb_v,
                  ls0, ls1, as0, as1):
    c = lax.axis_index("c")
    s = lax.axis_index("s")
    base_node = c * HC
    idx = (idx0_v, idx1_v)
    rlo = (rlo0_v, rlo1_v)
    rhi = (rhi0_v, rhi1_v)
    lsem = (ls0, ls1)
    asem = (as0, as1)

    _zero_fill(zb_v, 40, H)
    stripe = pl.multiple_of(s * 328, 8)
    for acc in (alo_sh, ahi_sh):
        for q in range(8):
            pltpu.sync_copy(zb_v, acc.at[pl.ds(stripe + q * 40, 40)])
        pltpu.sync_copy(zb_v.at[pl.ds(0, 8)], acc.at[pl.ds(stripe + 320, 8)])

    plsc.subcore_barrier()

    ept = E // NS
    rbase = pl.multiple_of(s * ept, 8)

    def do_array(msg_hbm, idx_hbm):
        def lstart(b, j):
            off = pl.multiple_of(j * CS, 8)
            pltpu.async_copy(idx_hbm.at[pl.ds(rbase + off, CS)],
                             idx[b], lsem[b])
            pltpu.async_copy(msg_hbm.at[pl.ds(rbase + off, CS), pl.ds(0, H)],
                             rlo[b], lsem[b])
            pltpu.async_copy(msg_hbm.at[pl.ds(rbase + off, CS), pl.ds(H, H)],
                             rhi[b], lsem[b])

        def lwait(b):
            pltpu.make_async_copy(idx_hbm.at[pl.ds(rbase, CS)],
                                  idx[b], lsem[b]).wait()
            pltpu.make_async_copy(
                msg_hbm.at[pl.ds(rbase, CS), pl.ds(0, H)],
                rlo[b], lsem[b]).wait()
            pltpu.make_async_copy(
                msg_hbm.at[pl.ds(rbase, CS), pl.ds(H, H)],
                rhi[b], lsem[b]).wait()

        def astart(b):
            pltpu.async_copy(rlo[b], alo_sh.at[idx[b]], asem[b], add=True)
            pltpu.async_copy(rhi[b], ahi_sh.at[idx[b]], asem[b], add=True)

        def await_(b):
            pltpu.make_async_copy(rlo[b], alo_sh.at[idx[b]], asem[b]).wait()
            pltpu.make_async_copy(rhi[b], ahi_sh.at[idx[b]], asem[b]).wait()

        lstart(0, 0)

        def pair(p, _):
            lwait(0)
            _localize(idx0_v, base_node, 128)

            @pl.when(p > 0)
            def _():
                await_(1)

            lstart(1, 2 * p + 1)
            astart(0)
            lwait(1)
            _localize(idx1_v, base_node, 128)
            await_(0)
            lstart(0, 2 * p + 2)
            astart(1)
            return 0

        lax.fori_loop(0, (SCH - 1) // 2, pair, 0)
        lwait(0)
        _localize(idx0_v, base_node, 128)
        await_(1)
        astart(0)
        await_(0)

    do_array(mi_hbm, dst_hbm)
    do_array(mo_hbm, src_hbm)

    plsc.subcore_barrier()

    dump = pl.multiple_of(s * 320, 8)
    obase = pl.multiple_of(c * HC + s * 320, 8)
    pltpu.sync_copy(alo_sh.at[pl.ds(dump, 320)],
                    s_hbm.at[pl.ds(obase, 320), pl.ds(0, H)])
    pltpu.sync_copy(ahi_sh.at[pl.ds(dump, 320)],
                    s_hbm.at[pl.ds(obase, 320), pl.ds(H, H)])


def _scatter(mi, mo, src, dst):
    f = functools.partial(
        pl.kernel,
        mesh=_sc_mesh(),
        out_type=jax.ShapeDtypeStruct((SOUT, D), jnp.float32),
        scratch_types=[
            pltpu.VMEM_SHARED((ACC_R, H), jnp.float32),
            pltpu.VMEM_SHARED((ACC_R, H), jnp.float32),
            pltpu.VMEM((CS,), jnp.int32),
            pltpu.VMEM((CS,), jnp.int32),
            pltpu.VMEM((CS, H), jnp.float32),
            pltpu.VMEM((CS, H), jnp.float32),
            pltpu.VMEM((CS, H), jnp.float32),
            pltpu.VMEM((CS, H), jnp.float32),
            pltpu.VMEM((40, H), jnp.float32),
            pltpu.SemaphoreType.DMA,
            pltpu.SemaphoreType.DMA,
            pltpu.SemaphoreType.DMA,
            pltpu.SemaphoreType.DMA,
        ],
    )(_scatter_body)
    return f(mi, mo, src, dst)


DACC_R = 5128


def _degree_body(src_hbm, dst_hbm, deg_hbm, dacc_sh,
                 di0_v, di1_v, ones_v, zb_v, ls0, ls1, as0, as1):
    c = lax.axis_index("c")
    s = lax.axis_index("s")
    base_node = c * HC

    _zero_fill(zb_v, 40, 128)
    stripe = pl.multiple_of(s * 320, 8)
    for q in range(8):
        pltpu.sync_copy(zb_v, dacc_sh.at[pl.ds(stripe + q * 40, 40)])

    @pl.when(s == 0)
    def _():
        pltpu.sync_copy(zb_v.at[pl.ds(0, 8)], dacc_sh.at[pl.ds(HC, 8)])

    def orow(r, _):
        for k in range(128 // 16):
            ones_v[r, pl.ds(k * 16, 16)] = jnp.ones((16,), jnp.float32)
        return 0
    lax.fori_loop(0, CS, orow, 0)

    plsc.subcore_barrier()

    ept = E // NS
    rbase = pl.multiple_of(s * ept, 8)
    di = (di0_v, di1_v)
    lsem = (ls0, ls1)
    asem = (as0, as1)

    def do_array(idx_hbm):
        def lstart(b, j):
            off = pl.multiple_of(j * CS, 8)
            pltpu.async_copy(idx_hbm.at[pl.ds(rbase + off, CS)],
                             di[b], lsem[b])

        def lwait(b):
            pltpu.make_async_copy(idx_hbm.at[pl.ds(rbase, CS)],
                                  di[b], lsem[b]).wait()

        def astart(b):
            pltpu.async_copy(ones_v, dacc_sh.at[di[b]], asem[b], add=True)

        def await_(b):
            pltpu.make_async_copy(ones_v, dacc_sh.at[di[b]], asem[b]).wait()

        lstart(0, 0)

        def pair(p, _):
            lwait(0)
            _localize(di0_v, base_node, 8)

            @pl.when(p > 0)
            def _():
                await_(1)

            lstart(1, 2 * p + 1)
            astart(0)
            lwait(1)
            _localize(di1_v, base_node, 8)
            await_(0)
            lstart(0, 2 * p + 2)
            astart(1)
            return 0

        lax.fori_loop(0, (SCH - 1) // 2, pair, 0)
        lwait(0)
        _localize(di0_v, base_node, 8)
        await_(1)
        astart(0)
        await_(0)

    do_array(dst_hbm)
    do_array(src_hbm)

    plsc.subcore_barrier()

    obase = pl.multiple_of(c * HC + stripe, 8)
    pltpu.sync_copy(dacc_sh.at[pl.ds(stripe, 320)],
                    deg_hbm.at[pl.ds(obase, 320)])


def _degree(src, dst):
    f = functools.partial(
        pl.kernel,
        mesh=_sc_mesh(),
        out_type=jax.ShapeDtypeStruct((NPAD, 128), jnp.float32),
        scratch_types=[
            pltpu.VMEM_SHARED((DACC_R, 128), jnp.float32),
            pltpu.VMEM((CS,), jnp.int32),
            pltpu.VMEM((CS,), jnp.int32),
            pltpu.VMEM((CS, 128), jnp.float32),
            pltpu.VMEM((40, 128), jnp.float32),
            pltpu.SemaphoreType.DMA,
            pltpu.SemaphoreType.DMA,
            pltpu.SemaphoreType.DMA,
            pltpu.SemaphoreType.DMA,
        ],
    )(_degree_body)
    return f(src, dst)


def _k1_body(xu_ref, xv_ref, fiw_ref, fow_ref,
             h1i_ref, h1o_ref, si_ref, qi_ref, so_ref, qo_ref):
    xu = xu_ref[...].astype(jnp.bfloat16)
    xv = xv_ref[...].astype(jnp.bfloat16)
    h1i = (jnp.dot(xu, fiw_ref[:D, :], preferred_element_type=jnp.float32)
           + jnp.dot(xv, fiw_ref[D:, :], preferred_element_type=jnp.float32))
    h1o = (jnp.dot(xv, fow_ref[:D, :], preferred_element_type=jnp.float32)
           + jnp.dot(xu, fow_ref[D:, :], preferred_element_type=jnp.float32))
    h1i_ref[...] = h1i.astype(jnp.bfloat16)
    h1o_ref[...] = h1o.astype(jnp.bfloat16)

    @pl.when(pl.program_id(0) == 0)
    def _():
        si_ref[...] = jnp.zeros_like(si_ref)
        qi_ref[...] = jnp.zeros_like(qi_ref)
        so_ref[...] = jnp.zeros_like(so_ref)
        qo_ref[...] = jnp.zeros_like(qo_ref)

    si_ref[...] += jnp.sum(h1i, axis=0, keepdims=True)
    qi_ref[...] += jnp.sum(h1i * h1i, axis=0, keepdims=True)
    so_ref[...] += jnp.sum(h1o, axis=0, keepdims=True)
    qo_ref[...] += jnp.sum(h1o * h1o, axis=0, keepdims=True)


def _affine(sum_ref, sq_ref, g_ref, be_ref, n):
    m = sum_ref[...] * (1.0 / n)
    v = sq_ref[...] * (1.0 / n) - m * m
    a = g_ref[...] * lax.rsqrt(v + EPS)
    return a, be_ref[...] - m * a


def _k2_body(h1i_ref, h1o_ref, fiw2_ref, fow2_ref,
             si_ref, qi_ref, so_ref, qo_ref,
             gi_ref, bi_ref, go_ref, bo_ref,
             h2i_ref, h2o_ref, s2i_ref, q2i_ref, s2o_ref, q2o_ref):
    ai, ci = _affine(si_ref, qi_ref, gi_ref, bi_ref, float(E))
    ao, co = _affine(so_ref, qo_ref, go_ref, bo_ref, float(E))
    x2i = jnp.maximum(h1i_ref[...].astype(jnp.float32) * ai + ci,
                      0.0).astype(jnp.bfloat16)
    x2o = jnp.maximum(h1o_ref[...].astype(jnp.float32) * ao + co,
                      0.0).astype(jnp.bfloat16)
    h2i = jnp.dot(x2i, fiw2_ref[...], preferred_element_type=jnp.float32)
    h2o = jnp.dot(x2o, fow2_ref[...], preferred_element_type=jnp.float32)
    h2i_ref[...] = h2i.astype(jnp.bfloat16)
    h2o_ref[...] = h2o.astype(jnp.bfloat16)

    @pl.when(pl.program_id(0) == 0)
    def _():
        s2i_ref[...] = jnp.zeros_like(s2i_ref)
        q2i_ref[...] = jnp.zeros_like(q2i_ref)
        s2o_ref[...] = jnp.zeros_like(s2o_ref)
        q2o_ref[...] = jnp.zeros_like(q2o_ref)

    s2i_ref[...] += jnp.sum(h2i, axis=0, keepdims=True)
    q2i_ref[...] += jnp.sum(h2i * h2i, axis=0, keepdims=True)
    s2o_ref[...] += jnp.sum(h2o, axis=0, keepdims=True)
    q2o_ref[...] += jnp.sum(h2o * h2o, axis=0, keepdims=True)


def _k3_body(h2i_ref, h2o_ref,
             s2i_ref, q2i_ref, s2o_ref, q2o_ref,
             gi_ref, bi_ref, go_ref, bo_ref,
             mi_ref, mo_ref):
    ai, ci = _affine(s2i_ref, q2i_ref, gi_ref, bi_ref, float(E))
    ao, co = _affine(s2o_ref, q2o_ref, go_ref, bo_ref, float(E))
    mi_ref[...] = jnp.maximum(h2i_ref[...].astype(jnp.float32) * ai + ci, 0.0)
    mo_ref[...] = jnp.maximum(h2o_ref[...].astype(jnp.float32) * ao + co, 0.0)


def _k4_body(x_ref, s_ref, deg_ref, fpw_ref, g_ref, sp_ref, qp_ref):
    dv = jnp.maximum(deg_ref[...][:, 0:1], 1.0)
    h = (x_ref[...] + s_ref[...] * (1.0 / dv)).astype(jnp.bfloat16)
    g = jnp.dot(h, fpw_ref[...], preferred_element_type=jnp.float32)
    g_ref[...] = g

    @pl.when(pl.program_id(0) == 0)
    def _():
        sp_ref[...] = jnp.zeros_like(sp_ref)
        qp_ref[...] = jnp.zeros_like(qp_ref)

    sp_ref[...] += jnp.sum(g, axis=0, keepdims=True)
    qp_ref[...] += jnp.sum(g * g, axis=0, keepdims=True)


def _k5_body(g_ref, sp_ref, qp_ref, gg_ref, bb_ref, x_ref):
    a, c = _affine(sp_ref, qp_ref, gg_ref, bb_ref, float(N))
    x_ref[...] = jnp.maximum(g_ref[...] * a + c, 0.0)


def _row_spec(tb, d):
    return pl.BlockSpec((tb, d), lambda i: (i, 0))


def _full_spec(shape):
    return pl.BlockSpec(shape, lambda i: tuple(0 for _ in shape))


_STAT = _full_spec((1, D))


def _k1(xu, xv, fiw1, fow1):
    ge = E // TBE
    return pl.pallas_call(
        _k1_body,
        grid=(ge,),
        in_specs=[_row_spec(TBE, D), _row_spec(TBE, D),
                  _full_spec((2 * D, D)), _full_spec((2 * D, D))],
        out_specs=[_row_spec(TBE, D), _row_spec(TBE, D),
                   _STAT, _STAT, _STAT, _STAT],
        out_shape=[jax.ShapeDtypeStruct((E, D), jnp.bfloat16),
                   jax.ShapeDtypeStruct((E, D), jnp.bfloat16)]
                  + [jax.ShapeDtypeStruct((1, D), jnp.float32)] * 4,
    )(xu, xv, fiw1, fow1)


def _k2(h1i, h1o, fiw2, fow2, si, qi, so, qo, gi, bi, go, bo):
    ge = E // TBE
    return pl.pallas_call(
        _k2_body,
        grid=(ge,),
        in_specs=[_row_spec(TBE, D), _row_spec(TBE, D),
                  _full_spec((D, D)), _full_spec((D, D))]
                 + [_STAT] * 8,
        out_specs=[_row_spec(TBE, D), _row_spec(TBE, D),
                   _STAT, _STAT, _STAT, _STAT],
        out_shape=[jax.ShapeDtypeStruct((E, D), jnp.bfloat16),
                   jax.ShapeDtypeStruct((E, D), jnp.bfloat16)]
                  + [jax.ShapeDtypeStruct((1, D), jnp.float32)] * 4,
    )(h1i, h1o, fiw2, fow2, si, qi, so, qo, gi, bi, go, bo)


def _k3(h2i, h2o, s2i, q2i, s2o, q2o, gi, bi, go, bo):
    ge = E // TBE
    return pl.pallas_call(
        _k3_body,
        grid=(ge,),
        in_specs=[_row_spec(TBE, D), _row_spec(TBE, D)] + [_STAT] * 8,
        out_specs=[_row_spec(TBE, D), _row_spec(TBE, D)],
        out_shape=[jax.ShapeDtypeStruct((E, D), jnp.float32),
                   jax.ShapeDtypeStruct((E, D), jnp.float32)],
    )(h2i, h2o, s2i, q2i, s2o, q2o, gi, bi, go, bo)


def _k4(x, s, deg, fpw):
    gn = N // TBN
    return pl.pallas_call(
        _k4_body,
        grid=(gn,),
        in_specs=[_row_spec(TBN, D), _row_spec(TBN, D),
                  _row_spec(TBN, 128), _full_spec((D, D))],
        out_specs=[_row_spec(TBN, D), _STAT, _STAT],
        out_shape=[jax.ShapeDtypeStruct((N, D), jnp.float32),
                   jax.ShapeDtypeStruct((1, D), jnp.float32),
                   jax.ShapeDtypeStruct((1, D), jnp.float32)],
    )(x, s, deg, fpw)


def _k5(g, sp, qp, gg, bb):
    gn = N // TBN
    return pl.pallas_call(
        _k5_body,
        grid=(gn,),
        in_specs=[_row_spec(TBN, D)] + [_STAT] * 4,
        out_specs=_row_spec(TBN, D),
        out_shape=jax.ShapeDtypeStruct((N, D), jnp.float32),
    )(g, sp, qp, gg, bb)


def kernel(x, edge_index,
           fi_w1, fi_b1, fi_g1, fi_be1, fi_w2, fi_b2, fi_g2, fi_be2,
           fo_w1, fo_b1, fo_g1, fo_be1, fo_w2, fo_b2, fo_g2, fo_be2,
           fp_w1, fp_b1, fp_g1, fp_be1):
    del fi_b1, fi_b2, fo_b1, fo_b2, fp_b1
    r = lambda t: t.reshape(1, D)
    gi1, bi1, gi2, bi2 = r(fi_g1), r(fi_be1), r(fi_g2), r(fi_be2)
    go1, bo1, go2, bo2 = r(fo_g1), r(fo_be1), r(fo_g2), r(fo_be2)
    gp, bp = r(fp_g1), r(fp_be1)
    fi_w1 = fi_w1.astype(jnp.bfloat16)
    fo_w1 = fo_w1.astype(jnp.bfloat16)
    fi_w2 = fi_w2.astype(jnp.bfloat16)
    fo_w2 = fo_w2.astype(jnp.bfloat16)
    fp_w1 = fp_w1.astype(jnp.bfloat16)

    src, dst = edge_index[0], edge_index[1]
    deg = _degree(src, dst)[:N]
    for step in range(2):
        xu, xv = _gather(x, src, dst)
        h1i, h1o, si, qi, so, qo = _k1(xu, xv, fi_w1, fo_w1)
        h2i, h2o, s2i, q2i, s2o, q2o = _k2(
            h1i, h1o, fi_w2, fo_w2, si, qi, so, qo, gi1, bi1, go1, bo1)
        mi, mo = _k3(h2i, h2o, s2i, q2i, s2o, q2o, gi2, bi2, go2, bo2)
        s_agg = _scatter(mi, mo, src, dst)
        g, sp, qp = _k4(x, s_agg[:N], deg, fp_w1)
        x = _k5(g, sp, qp, gp, bp)
    return x


# speedup vs baseline: 2.4195x; 1.0980x over previous
import functools

import jax
import jax.numpy as jnp
from jax import lax
from jax.experimental import pallas as pl
from jax.experimental.pallas import tpu as pltpu
from jax.experimental.pallas import tpu_sc as plsc

N = 10000
E = 160000
D = 256
EPS = 1e-5

NC = 2
NS = 16
NW = NC * NS

HC = 5120
ACC_R = 5248
NPAD = 2 * HC

CG = 40
GCH = (E // NW) // CG
CS = 80
SCH = (E // NS) // CS

TBE = 2000
TBN = 2000

def _sc_mesh():
    return plsc.VectorSubcoreMesh(core_axis_name="c", subcore_axis_name="s")


GNB = 5


def _gather_body(x_hbm, src_hbm, dst_hbm, xu_hbm, xv_hbm, idx_v, *refs):
    rows = refs[:GNB]
    gsem = refs[GNB:2 * GNB]
    wsem = refs[2 * GNB:3 * GNB]
    wid = lax.axis_index("s") * NC + lax.axis_index("c")
    epw = E // NW
    base = pl.multiple_of(wid * epw, epw)

    for idx_hbm, out in ((src_hbm, xu_hbm), (dst_hbm, xv_hbm)):
        pltpu.sync_copy(idx_hbm.at[pl.ds(base, epw)], idx_v)

        def gstart(b, j):
            off = pl.multiple_of(j * CG, 8)
            pltpu.async_copy(x_hbm.at[idx_v.at[pl.ds(off, CG)]],
                             rows[b], gsem[b])

        def gwait(b):
            pltpu.make_async_copy(x_hbm.at[idx_v.at[pl.ds(0, CG)]],
                                  rows[b], gsem[b]).wait()

        def wstart(b, j):
            off = pl.multiple_of(j * CG, 8)
            pltpu.async_copy(rows[b], out.at[pl.ds(base + off, CG)], wsem[b])

        def wwait(b):
            pltpu.make_async_copy(rows[b], out.at[pl.ds(base, CG)],
                                  wsem[b]).wait()

        for b in range(GNB - 1):
            gstart(b, b)

        def round_(p, _):
            for q in range(GNB):
                g = GNB * p + q
                t = (q + GNB - 1) % GNB
                gwait(q)
                if q == 0:
                    @pl.when(p > 0)
                    def _():
                        wwait(t)
                else:
                    wwait(t)
                if q == 0:
                    gstart(t, g + GNB - 1)
                else:
                    @pl.when(p < GCH // GNB - 1)
                    def _():
                        gstart(t, g + GNB - 1)
                wstart(q, g)
            return 0

        lax.fori_loop(0, GCH // GNB, round_, 0)
        wwait(GNB - 1)


def _gather(x, src, dst):
    f = functools.partial(
        pl.kernel,
        mesh=_sc_mesh(),
        out_type=[
            jax.ShapeDtypeStruct((E, D), jnp.float32),
            jax.ShapeDtypeStruct((E, D), jnp.float32),
        ],
        scratch_types=(
            [pltpu.VMEM((E // NW,), jnp.int32)]
            + [pltpu.VMEM((CG, D), jnp.float32)] * GNB
            + [pltpu.SemaphoreType.DMA] * (2 * GNB)
        ),
    )(_gather_body)
    return f(x, src, dst)


def _zero_fill(buf, nrows, width):
    def zrow(r, _):
        for k in range(width // 16):
            buf[r, pl.ds(k * 16, 16)] = jnp.zeros((16,), jnp.float32)
        return 0
    lax.fori_loop(0, nrows, zrow, 0)


def _localize(idx_v, base_node, spread):
    it = lax.iota(jnp.int32, 16)
    for k in range(CS // 16):
        v = idx_v[pl.ds(k * 16, 16)]
        loc = v - base_node
        ok = (loc >= 0) & (loc < HC)
        trash = HC + ((it + k * 16) & (spread - 1))
        idx_v[pl.ds(k * 16, 16)] = jnp.where(ok, loc, trash)


SOUT = NPAD
H = 128


def _scatter_body(mi_hbm, mo_hbm, src_hbm, dst_hbm,
                  s_hbm, alo_sh, ahi_sh,
                  idx0_v, idx1_v, rlo0_v, rlo1_v, rhi0_v, rhi1_v, zb_v,
                  ls0, ls1, as0, as1):
    c = lax.axis_index("c")
    s = lax.axis_index("s")
    base_node = c * HC
    idx = (idx0_v, idx1_v)
    rlo = (rlo0_v, rlo1_v)
    rhi = (rhi0_v, rhi1_v)
    lsem = (ls0, ls1)
    asem = (as0, as1)

    _zero_fill(zb_v, 40, H)
    stripe = pl.multiple_of(s * 328, 8)
    for acc in (alo_sh, ahi_sh):
        for q in range(8):
            pltpu.sync_copy(zb_v, acc.at[pl.ds(stripe + q * 40, 40)])
        pltpu.sync_copy(zb_v.at[pl.ds(0, 8)], acc.at[pl.ds(stripe + 320, 8)])

    plsc.subcore_barrier()

    ept = E // NS
    rbase = pl.multiple_of(s * ept, 8)

    def do_array(msg_hbm, idx_hbm):
        def lstart(b, j):
            off = pl.multiple_of(j * CS, 8)
            pltpu.async_copy(idx_hbm.at[pl.ds(rbase + off, CS)],
                             idx[b], lsem[b])
            pltpu.async_copy(msg_hbm.at[pl.ds(rbase + off, CS), pl.ds(0, H)],
                             rlo[b], lsem[b])
            pltpu.async_copy(msg_hbm.at[pl.ds(rbase + off, CS), pl.ds(H, H)],
                             rhi[b], lsem[b])

        def lwait(b):
            pltpu.make_async_copy(idx_hbm.at[pl.ds(rbase, CS)],
                                  idx[b], lsem[b]).wait()
            pltpu.make_async_copy(
                msg_hbm.at[pl.ds(rbase, CS), pl.ds(0, H)],
                rlo[b], lsem[b]).wait()
            pltpu.make_async_copy(
                msg_hbm.at[pl.ds(rbase, CS), pl.ds(H, H)],
                rhi[b], lsem[b]).wait()

        def astart(b):
            pltpu.async_copy(rlo[b], alo_sh.at[idx[b]], asem[b], add=True)
            pltpu.async_copy(rhi[b], ahi_sh.at[idx[b]], asem[b], add=True)

        def await_(b):
            pltpu.make_async_copy(rlo[b], alo_sh.at[idx[b]], asem[b]).wait()
            pltpu.make_async_copy(rhi[b], ahi_sh.at[idx[b]], asem[b]).wait()

        lstart(0, 0)

        def pair(p, _):
            lwait(0)
            _localize(idx0_v, base_node, 128)

            @pl.when(p > 0)
            def _():
                await_(1)

            lstart(1, 2 * p + 1)
            astart(0)
            lwait(1)
            _localize(idx1_v, base_node, 128)
            await_(0)
            lstart(0, 2 * p + 2)
            astart(1)
            return 0

        lax.fori_loop(0, (SCH - 1) // 2, pair, 0)
        lwait(0)
        _localize(idx0_v, base_node, 128)
        await_(1)
        astart(0)
        await_(0)

    do_array(mi_hbm, dst_hbm)
    do_array(mo_hbm, src_hbm)

    plsc.subcore_barrier()

    dump = pl.multiple_of(s * 320, 8)
    obase = pl.multiple_of(c * HC + s * 320, 8)
    pltpu.sync_copy(alo_sh.at[pl.ds(dump, 320)],
                    s_hbm.at[pl.ds(obase, 320), pl.ds(0, H)])
    pltpu.sync_copy(ahi_sh.at[pl.ds(dump, 320)],
                    s_hbm.at[pl.ds(obase, 320), pl.ds(H, H)])


def _scatter(mi, mo, src, dst):
    f = functools.partial(
        pl.kernel,
        mesh=_sc_mesh(),
        out_type=jax.ShapeDtypeStruct((SOUT, D), jnp.float32),
        scratch_types=[
            pltpu.VMEM_SHARED((ACC_R, H), jnp.float32),
            pltpu.VMEM_SHARED((ACC_R, H), jnp.float32),
            pltpu.VMEM((CS,), jnp.int32),
            pltpu.VMEM((CS,), jnp.int32),
            pltpu.VMEM((CS, H), jnp.float32),
            pltpu.VMEM((CS, H), jnp.float32),
            pltpu.VMEM((CS, H), jnp.float32),
            pltpu.VMEM((CS, H), jnp.float32),
            pltpu.VMEM((40, H), jnp.float32),
            pltpu.SemaphoreType.DMA,
            pltpu.SemaphoreType.DMA,
            pltpu.SemaphoreType.DMA,
            pltpu.SemaphoreType.DMA,
        ],
    )(_scatter_body)
    return f(mi, mo, src, dst)


DACC_R = 5128


def _degree_body(src_hbm, dst_hbm, deg_hbm, dacc_sh,
                 di0_v, di1_v, ones_v, zb_v, ls0, ls1, as0, as1):
    c = lax.axis_index("c")
    s = lax.axis_index("s")
    base_node = c * HC

    _zero_fill(zb_v, 40, 128)
    stripe = pl.multiple_of(s * 320, 8)
    for q in range(8):
        pltpu.sync_copy(zb_v, dacc_sh.at[pl.ds(stripe + q * 40, 40)])

    @pl.when(s == 0)
    def _():
        pltpu.sync_copy(zb_v.at[pl.ds(0, 8)], dacc_sh.at[pl.ds(HC, 8)])

    def orow(r, _):
        for k in range(128 // 16):
            ones_v[r, pl.ds(k * 16, 16)] = jnp.ones((16,), jnp.float32)
        return 0
    lax.fori_loop(0, CS, orow, 0)

    plsc.subcore_barrier()

    ept = E // NS
    rbase = pl.multiple_of(s * ept, 8)
    di = (di0_v, di1_v)
    lsem = (ls0, ls1)
    asem = (as0, as1)

    def do_array(idx_hbm):
        def lstart(b, j):
            off = pl.multiple_of(j * CS, 8)
            pltpu.async_copy(idx_hbm.at[pl.ds(rbase + off, CS)],
                             di[b], lsem[b])

        def lwait(b):
            pltpu.make_async_copy(idx_hbm.at[pl.ds(rbase, CS)],
                                  di[b], lsem[b]).wait()

        def astart(b):
            pltpu.async_copy(ones_v, dacc_sh.at[di[b]], asem[b], add=True)

        def await_(b):
            pltpu.make_async_copy(ones_v, dacc_sh.at[di[b]], asem[b]).wait()

        lstart(0, 0)

        def pair(p, _):
            lwait(0)
            _localize(di0_v, base_node, 8)

            @pl.when(p > 0)
            def _():
                await_(1)

            lstart(1, 2 * p + 1)
            astart(0)
            lwait(1)
            _localize(di1_v, base_node, 8)
            await_(0)
            lstart(0, 2 * p + 2)
            astart(1)
            return 0

        lax.fori_loop(0, (SCH - 1) // 2, pair, 0)
        lwait(0)
        _localize(di0_v, base_node, 8)
        await_(1)
        astart(0)
        await_(0)

    do_array(dst_hbm)
    do_array(src_hbm)

    plsc.subcore_barrier()

    obase = pl.multiple_of(c * HC + stripe, 8)
    pltpu.sync_copy(dacc_sh.at[pl.ds(stripe, 320)],
                    deg_hbm.at[pl.ds(obase, 320)])


def _degree(src, dst):
    f = functools.partial(
        pl.kernel,
        mesh=_sc_mesh(),
        out_type=jax.ShapeDtypeStruct((NPAD, 128), jnp.float32),
        scratch_types=[
            pltpu.VMEM_SHARED((DACC_R, 128), jnp.float32),
            pltpu.VMEM((CS,), jnp.int32),
            pltpu.VMEM((CS,), jnp.int32),
            pltpu.VMEM((CS, 128), jnp.float32),
            pltpu.VMEM((40, 128), jnp.float32),
            pltpu.SemaphoreType.DMA,
            pltpu.SemaphoreType.DMA,
            pltpu.SemaphoreType.DMA,
            pltpu.SemaphoreType.DMA,
        ],
    )(_degree_body)
    return f(src, dst)


def _k1_body(xu_ref, xv_ref, fiw_ref, fow_ref,
             h1i_ref, h1o_ref, si_ref, qi_ref, so_ref, qo_ref):
    xu = xu_ref[...].astype(jnp.bfloat16)
    xv = xv_ref[...].astype(jnp.bfloat16)
    h1i = (jnp.dot(xu, fiw_ref[:D, :], preferred_element_type=jnp.float32)
           + jnp.dot(xv, fiw_ref[D:, :], preferred_element_type=jnp.float32))
    h1o = (jnp.dot(xv, fow_ref[:D, :], preferred_element_type=jnp.float32)
           + jnp.dot(xu, fow_ref[D:, :], preferred_element_type=jnp.float32))
    h1i_ref[...] = h1i.astype(jnp.bfloat16)
    h1o_ref[...] = h1o.astype(jnp.bfloat16)

    @pl.when(pl.program_id(0) == 0)
    def _():
        si_ref[...] = jnp.zeros_like(si_ref)
        qi_ref[...] = jnp.zeros_like(qi_ref)
        so_ref[...] = jnp.zeros_like(so_ref)
        qo_ref[...] = jnp.zeros_like(qo_ref)

    si_ref[...] += jnp.sum(h1i, axis=0, keepdims=True)
    qi_ref[...] += jnp.sum(h1i * h1i, axis=0, keepdims=True)
    so_ref[...] += jnp.sum(h1o, axis=0, keepdims=True)
    qo_ref[...] += jnp.sum(h1o * h1o, axis=0, keepdims=True)


def _affine(sum_ref, sq_ref, g_ref, be_ref, n):
    m = sum_ref[...] * (1.0 / n)
    v = sq_ref[...] * (1.0 / n) - m * m
    a = g_ref[...] * lax.rsqrt(v + EPS)
    return a, be_ref[...] - m * a


def _k2_body(h1i_ref, h1o_ref, fiw2_ref, fow2_ref,
             si_ref, qi_ref, so_ref, qo_ref,
             gi_ref, bi_ref, go_ref, bo_ref,
             h2i_ref, h2o_ref, s2i_ref, q2i_ref, s2o_ref, q2o_ref):
    ai, ci = _affine(si_ref, qi_ref, gi_ref, bi_ref, float(E))
    ao, co = _affine(so_ref, qo_ref, go_ref, bo_ref, float(E))
    x2i = jnp.maximum(h1i_ref[...].astype(jnp.float32) * ai + ci,
                      0.0).astype(jnp.bfloat16)
    x2o = jnp.maximum(h1o_ref[...].astype(jnp.float32) * ao + co,
                      0.0).astype(jnp.bfloat16)
    h2i = jnp.dot(x2i, fiw2_ref[...], preferred_element_type=jnp.float32)
    h2o = jnp.dot(x2o, fow2_ref[...], preferred_element_type=jnp.float32)
    h2i_ref[...] = h2i.astype(jnp.bfloat16)
    h2o_ref[...] = h2o.astype(jnp.bfloat16)

    @pl.when(pl.program_id(0) == 0)
    def _():
        s2i_ref[...] = jnp.zeros_like(s2i_ref)
        q2i_ref[...] = jnp.zeros_like(q2i_ref)
        s2o_ref[...] = jnp.zeros_like(s2o_ref)
        q2o_ref[...] = jnp.zeros_like(q2o_ref)

    s2i_ref[...] += jnp.sum(h2i, axis=0, keepdims=True)
    q2i_ref[...] += jnp.sum(h2i * h2i, axis=0, keepdims=True)
    s2o_ref[...] += jnp.sum(h2o, axis=0, keepdims=True)
    q2o_ref[...] += jnp.sum(h2o * h2o, axis=0, keepdims=True)


def _k3_body(h2i_ref, h2o_ref,
             s2i_ref, q2i_ref, s2o_ref, q2o_ref,
             gi_ref, bi_ref, go_ref, bo_ref,
             mi_ref, mo_ref):
    ai, ci = _affine(s2i_ref, q2i_ref, gi_ref, bi_ref, float(E))
    ao, co = _affine(s2o_ref, q2o_ref, go_ref, bo_ref, float(E))
    mi_ref[...] = jnp.maximum(h2i_ref[...].astype(jnp.float32) * ai + ci, 0.0)
    mo_ref[...] = jnp.maximum(h2o_ref[...].astype(jnp.float32) * ao + co, 0.0)


def _k4_body(x_ref, s_ref, deg_ref, fpw_ref, g_ref, sp_ref, qp_ref):
    dv = jnp.maximum(deg_ref[...][:, 0:1], 1.0)
    h = (x_ref[...] + s_ref[...] * (1.0 / dv)).astype(jnp.bfloat16)
    g = jnp.dot(h, fpw_ref[...], preferred_element_type=jnp.float32)
    g_ref[...] = g

    @pl.when(pl.program_id(0) == 0)
    def _():
        sp_ref[...] = jnp.zeros_like(sp_ref)
        qp_ref[...] = jnp.zeros_like(qp_ref)

    sp_ref[...] += jnp.sum(g, axis=0, keepdims=True)
    qp_ref[...] += jnp.sum(g * g, axis=0, keepdims=True)


def _k5_body(g_ref, sp_ref, qp_ref, gg_ref, bb_ref, x_ref):
    a, c = _affine(sp_ref, qp_ref, gg_ref, bb_ref, float(N))
    x_ref[...] = jnp.maximum(g_ref[...] * a + c, 0.0)


def _row_spec(tb, d):
    return pl.BlockSpec((tb, d), lambda i: (i, 0))


def _full_spec(shape):
    return pl.BlockSpec(shape, lambda i: tuple(0 for _ in shape))


_STAT = _full_spec((1, D))


def _k1(xu, xv, fiw1, fow1):
    ge = E // TBE
    return pl.pallas_call(
        _k1_body,
        grid=(ge,),
        in_specs=[_row_spec(TBE, D), _row_spec(TBE, D),
                  _full_spec((2 * D, D)), _full_spec((2 * D, D))],
        out_specs=[_row_spec(TBE, D), _row_spec(TBE, D),
                   _STAT, _STAT, _STAT, _STAT],
        out_shape=[jax.ShapeDtypeStruct((E, D), jnp.bfloat16),
                   jax.ShapeDtypeStruct((E, D), jnp.bfloat16)]
                  + [jax.ShapeDtypeStruct((1, D), jnp.float32)] * 4,
    )(xu, xv, fiw1, fow1)


def _k2(h1i, h1o, fiw2, fow2, si, qi, so, qo, gi, bi, go, bo):
    ge = E // TBE
    return pl.pallas_call(
        _k2_body,
        grid=(ge,),
        in_specs=[_row_spec(TBE, D), _row_spec(TBE, D),
                  _full_spec((D, D)), _full_spec((D, D))]
                 + [_STAT] * 8,
        out_specs=[_row_spec(TBE, D), _row_spec(TBE, D),
                   _STAT, _STAT, _STAT, _STAT],
        out_shape=[jax.ShapeDtypeStruct((E, D), jnp.bfloat16),
                   jax.ShapeDtypeStruct((E, D), jnp.bfloat16)]
                  + [jax.ShapeDtypeStruct((1, D), jnp.float32)] * 4,
    )(h1i, h1o, fiw2, fow2, si, qi, so, qo, gi, bi, go, bo)


def _k3(h2i, h2o, s2i, q2i, s2o, q2o, gi, bi, go, bo):
    ge = E // TBE
    return pl.pallas_call(
        _k3_body,
        grid=(ge,),
        in_specs=[_row_spec(TBE, D), _row_spec(TBE, D)] + [_STAT] * 8,
        out_specs=[_row_spec(TBE, D), _row_spec(TBE, D)],
        out_shape=[jax.ShapeDtypeStruct((E, D), jnp.float32),
                   jax.ShapeDtypeStruct((E, D), jnp.float32)],
    )(h2i, h2o, s2i, q2i, s2o, q2o, gi, bi, go, bo)


def _k4(x, s, deg, fpw):
    gn = N // TBN
    return pl.pallas_call(
        _k4_body,
        grid=(gn,),
        in_specs=[_row_spec(TBN, D), _row_spec(TBN, D),
                  _row_spec(TBN, 128), _full_spec((D, D))],
        out_specs=[_row_spec(TBN, D), _STAT, _STAT],
        out_shape=[jax.ShapeDtypeStruct((N, D), jnp.float32),
                   jax.ShapeDtypeStruct((1, D), jnp.float32),
                   jax.ShapeDtypeStruct((1, D), jnp.float32)],
    )(x, s, deg, fpw)


def _k5(g, sp, qp, gg, bb):
    gn = N // TBN
    return pl.pallas_call(
        _k5_body,
        grid=(gn,),
        in_specs=[_row_spec(TBN, D)] + [_STAT] * 4,
        out_specs=_row_spec(TBN, D),
        out_shape=jax.ShapeDtypeStruct((N, D), jnp.float32),
    )(g, sp, qp, gg, bb)


def kernel(x, edge_index,
           fi_w1, fi_b1, fi_g1, fi_be1, fi_w2, fi_b2, fi_g2, fi_be2,
           fo_w1, fo_b1, fo_g1, fo_be1, fo_w2, fo_b2, fo_g2, fo_be2,
           fp_w1, fp_b1, fp_g1, fp_be1):
    del fi_b1, fi_b2, fo_b1, fo_b2, fp_b1
    r = lambda t: t.reshape(1, D)
    gi1, bi1, gi2, bi2 = r(fi_g1), r(fi_be1), r(fi_g2), r(fi_be2)
    go1, bo1, go2, bo2 = r(fo_g1), r(fo_be1), r(fo_g2), r(fo_be2)
    gp, bp = r(fp_g1), r(fp_be1)
    fi_w1 = fi_w1.astype(jnp.bfloat16)
    fo_w1 = fo_w1.astype(jnp.bfloat16)
    fi_w2 = fi_w2.astype(jnp.bfloat16)
    fo_w2 = fo_w2.astype(jnp.bfloat16)
    fp_w1 = fp_w1.astype(jnp.bfloat16)

    src, dst = edge_index[0], edge_index[1]
    deg = _degree(src, dst)[:N]
    for step in range(2):
        xu, xv = _gather(x, src, dst)
        h1i, h1o, si, qi, so, qo = _k1(xu, xv, fi_w1, fo_w1)
        h2i, h2o, s2i, q2i, s2o, q2o = _k2(
            h1i, h1o, fi_w2, fo_w2, si, qi, so, qo, gi1, bi1, go1, bo1)
        mi, mo = _k3(h2i, h2o, s2i, q2i, s2o, q2o, gi2, bi2, go2, bo2)
        s_agg = _scatter(mi, mo, src, dst)
        g, sp, qp = _k4(x, s_agg[:N], deg, fp_w1)
        x = _k5(g, sp, qp, gp, bp)
    return x
